# Initial kernel scaffold; baseline (speedup 1.0000x reference)
import jax
import jax.numpy as jnp
from jax.experimental import pallas as pl


def kernel(x_chemical, x_disease, x_side_effect, edge_index_treats, edge_index_rev_treats, Wp_c, bp_c, Wp_d, bp_d, Wp_s, bp_s, Wl1_td, bl1_td, Wr1_td, Wl1_dc, bl1_dc, Wr1_dc, Wl2_td, bl2_td, Wr2_td, Wl2_dc, bl2_dc, Wr2_dc):
    raise NotImplementedError("write your pallas kernel here")


# speedup vs baseline: 7.0046x; 7.0046x over previous
import functools

import jax
import jax.numpy as jnp
from jax import lax
from jax.experimental import pallas as pl
from jax.experimental.pallas import tpu as pltpu
from jax.experimental.pallas import tpu_sc as plsc

N_NODES = 10000
IN_DIM = 128
D = 64
CW = 16
E = 640000

NC = 2
NS = 16
NW = NC * NS
EPW = E // NW
CHUNK = 80
NCHUNK = EPW // CHUNK

_ZROWS = [640] * 15 + [400]
_ZOFF = [640 * i for i in range(16)]


def _sc_agg_body(with_counts, src_a, dst_a, tab_a, src_b, dst_b, tab_b,
                 zeros64, zeros16, ones_hbm,
                 *refs):
    if with_counts:
        (agg_a_out, agg_b_out, cnt_a_out, cnt_b_out,
         idx_s, idx_d, rows, ones_v, acc_a, acc_b, cacc_a, cacc_b,
         sem) = refs
    else:
        (agg_a_out, agg_b_out,
         idx_s, idx_d, rows, acc_a, acc_b, sem) = refs

    c = lax.axis_index("c")
    s = lax.axis_index("s")
    wid = s * NC + c

    for t in range(NS):
        @pl.when(s == t)
        def _():
            sl = pl.ds(_ZOFF[t], _ZROWS[t])
            pltpu.sync_copy(zeros64.at[sl], acc_a.at[sl])
            pltpu.sync_copy(zeros64.at[sl], acc_b.at[sl])
            if with_counts:
                pltpu.sync_copy(zeros16.at[sl], cacc_a.at[sl])
                pltpu.sync_copy(zeros16.at[sl], cacc_b.at[sl])

    if with_counts:
        pltpu.sync_copy(ones_hbm, ones_v)

    plsc.subcore_barrier()

    base = wid * EPW

    def _do_direction(src_hbm, dst_hbm, tab_hbm, acc, cacc):
        def step(i, _):
            off = base + i * CHUNK
            pltpu.sync_copy(src_hbm.at[pl.ds(off, CHUNK)], idx_s)
            pltpu.sync_copy(dst_hbm.at[pl.ds(off, CHUNK)], idx_d)
            pltpu.async_copy(tab_hbm.at[idx_s], rows, sem).wait()
            pltpu.sync_copy(rows, acc.at[idx_d], add=True)
            if with_counts:
                pltpu.sync_copy(ones_v, cacc.at[idx_d], add=True)
            return 0
        lax.fori_loop(0, NCHUNK, step, 0)

    _do_direction(src_a, dst_a, tab_a, acc_a, cacc_a if with_counts else None)
    _do_direction(src_b, dst_b, tab_b, acc_b, cacc_b if with_counts else None)

    plsc.subcore_barrier()

    for t in range(NS):
        @pl.when(s == t)
        def _():
            sl = pl.ds(_ZOFF[t], _ZROWS[t])
            pltpu.sync_copy(acc_a.at[sl], agg_a_out.at[c, sl])
            pltpu.sync_copy(acc_b.at[sl], agg_b_out.at[c, sl])
            if with_counts:
                pltpu.sync_copy(cacc_a.at[sl], cnt_a_out.at[c, sl])
                pltpu.sync_copy(cacc_b.at[sl], cnt_b_out.at[c, sl])


def _make_sc_agg(with_counts):
    mesh = plsc.VectorSubcoreMesh(core_axis_name="c", subcore_axis_name="s",
                                  num_cores=NC, num_subcores=NS)
    out_type = [
        jax.ShapeDtypeStruct((NC, N_NODES, D), jnp.float32),
        jax.ShapeDtypeStruct((NC, N_NODES, D), jnp.float32),
    ]
    scratch = [
        pltpu.VMEM((CHUNK,), jnp.int32),
        pltpu.VMEM((CHUNK,), jnp.int32),
        pltpu.VMEM((CHUNK, D), jnp.float32),
    ]
    if with_counts:
        out_type += [
            jax.ShapeDtypeStruct((NC, N_NODES, CW), jnp.float32),
            jax.ShapeDtypeStruct((NC, N_NODES, CW), jnp.float32),
        ]
        scratch += [pltpu.VMEM((CHUNK, CW), jnp.float32)]
    scratch += [
        pltpu.VMEM_SHARED((N_NODES, D), jnp.float32),
        pltpu.VMEM_SHARED((N_NODES, D), jnp.float32),
    ]
    if with_counts:
        scratch += [
            pltpu.VMEM_SHARED((N_NODES, CW), jnp.float32),
            pltpu.VMEM_SHARED((N_NODES, CW), jnp.float32),
        ]
    scratch += [pltpu.SemaphoreType.DMA]
    return pl.kernel(
        functools.partial(_sc_agg_body, with_counts),
        out_type=tuple(out_type),
        mesh=mesh,
        scratch_types=tuple(scratch),
        compiler_params=pltpu.CompilerParams(use_tc_tiling_on_sc=False),
    )


def _proj_body(xc, wc, bc, xd, wd, bd, xs, ws, bs, oc, od, os_):
    dn = (((1,), (1,)), ((), ()))
    oc[...] = lax.dot_general(xc[...], wc[...], dn,
                              preferred_element_type=jnp.float32) + bc[...]
    od[...] = lax.dot_general(xd[...], wd[...], dn,
                              preferred_element_type=jnp.float32) + bd[...]
    os_[...] = jnp.maximum(
        lax.dot_general(xs[...], ws[...], dn,
                        preferred_element_type=jnp.float32) + bs[...], 0.0)


def _conv_body(relu, aggp_a, cntp_a, xdst_a, wl_a, bl_a, wr_a,
               aggp_b, cntp_b, xdst_b, wl_b, bl_b, wr_b, oa, ob):
    dn = (((1,), (1,)), ((), ()))

    def one(aggp, cntp, xdst, wl, bl, wr, out):
        agg = aggp[0] + aggp[1]
        cnt = cntp[0, :, 0] + cntp[1, :, 0]
        inv = 1.0 / jnp.maximum(cnt, 1.0)
        mean = agg * inv[:, None]
        r = (lax.dot_general(mean, wl[...], dn,
                             preferred_element_type=jnp.float32) + bl[...] +
             lax.dot_general(xdst[...], wr[...], dn,
                             preferred_element_type=jnp.float32))
        out[...] = jnp.maximum(r, 0.0) if relu else r

    one(aggp_a, cntp_a, xdst_a, wl_a, bl_a, wr_a, oa)
    one(aggp_b, cntp_b, xdst_b, wl_b, bl_b, wr_b, ob)


_BM = 1000
_GRID = N_NODES // _BM


def _proj_call(xc, Wc, bc, xd, Wd, bd, xs, Ws, bs):
    xspec = pl.BlockSpec((_BM, IN_DIM), lambda m: (m, 0))
    wspec = pl.BlockSpec((D, IN_DIM), lambda m: (0, 0))
    bspec = pl.BlockSpec((1, D), lambda m: (0, 0))
    ospec = pl.BlockSpec((_BM, D), lambda m: (m, 0))
    oshape = jax.ShapeDtypeStruct((N_NODES, D), jnp.float32)
    return pl.pallas_call(
        _proj_body,
        grid=(_GRID,),
        in_specs=[xspec, wspec, bspec] * 3,
        out_specs=[ospec] * 3,
        out_shape=[oshape] * 3,
    )(xc, Wc, bc.reshape(1, D), xd, Wd, bd.reshape(1, D),
      xs, Ws, bs.reshape(1, D))


def _conv_call(relu, aggp_a, cntp_a, xdst_a, wl_a, bl_a, wr_a,
               aggp_b, cntp_b, xdst_b, wl_b, bl_b, wr_b):
    aspec = pl.BlockSpec((NC, _BM, D), lambda m: (0, m, 0))
    cspec = pl.BlockSpec((NC, _BM, CW), lambda m: (0, m, 0))
    xspec = pl.BlockSpec((_BM, D), lambda m: (m, 0))
    wspec = pl.BlockSpec((D, D), lambda m: (0, 0))
    bspec = pl.BlockSpec((1, D), lambda m: (0, 0))
    ospec = pl.BlockSpec((_BM, D), lambda m: (m, 0))
    oshape = jax.ShapeDtypeStruct((N_NODES, D), jnp.float32)
    return pl.pallas_call(
        functools.partial(_conv_body, relu),
        grid=(_GRID,),
        in_specs=[aspec, cspec, xspec, wspec, bspec, wspec] * 2,
        out_specs=[ospec] * 2,
        out_shape=[oshape] * 2,
    )(aggp_a, cntp_a, xdst_a, wl_a, bl_a.reshape(1, D), wr_a,
      aggp_b, cntp_b, xdst_b, wl_b, bl_b.reshape(1, D), wr_b)


def kernel(x_chemical, x_disease, x_side_effect, edge_index_treats,
           edge_index_rev_treats,
           Wp_c, bp_c, Wp_d, bp_d, Wp_s, bp_s,
           Wl1_td, bl1_td, Wr1_td, Wl1_dc, bl1_dc, Wr1_dc,
           Wl2_td, bl2_td, Wr2_td, Wl2_dc, bl2_dc, Wr2_dc):
    src_td = edge_index_treats[0]
    dst_td = edge_index_treats[1]
    src_dc = edge_index_rev_treats[0]
    dst_dc = edge_index_rev_treats[1]
    zeros64 = jnp.zeros((N_NODES, D), jnp.float32)
    zeros16 = jnp.zeros((N_NODES, CW), jnp.float32)
    ones = jnp.ones((CHUNK, CW), jnp.float32)

    xc, xd, s1 = _proj_call(x_chemical, Wp_c, bp_c, x_disease, Wp_d, bp_d,
                            x_side_effect, Wp_s, bp_s)

    sc1 = _make_sc_agg(True)
    aggp_td, aggp_dc, cntp_td, cntp_dc = sc1(
        src_td, dst_td, xc, src_dc, dst_dc, xd, zeros64, zeros16, ones)

    d1, c1 = _conv_call(True,
                        aggp_td, cntp_td, xd, Wl1_td, bl1_td, Wr1_td,
                        aggp_dc, cntp_dc, xc, Wl1_dc, bl1_dc, Wr1_dc)

    sc2 = _make_sc_agg(False)
    aggp2_td, aggp2_dc = sc2(
        src_td, dst_td, c1, src_dc, dst_dc, d1, zeros64, zeros16, ones)

    d2, c2 = _conv_call(False,
                        aggp2_td, cntp_td, d1, Wl2_td, bl2_td, Wr2_td,
                        aggp2_dc, cntp_dc, c1, Wl2_dc, bl2_dc, Wr2_dc)

    return c2, d2, s1


# speedup vs baseline: 19.4625x; 2.7785x over previous
import functools

import jax
import jax.numpy as jnp
from jax import lax
from jax.experimental import pallas as pl
from jax.experimental.pallas import tpu as pltpu
from jax.experimental.pallas import tpu_sc as plsc

N_NODES = 10000
IN_DIM = 128
D = 64
CW = 8
E = 640000

NC = 2
NS = 16
NW = NC * NS
EPW = E // NW
CHUNK = 80
NCHUNK = EPW // CHUNK

_ZROWS = [640] * 15 + [400]
_ZOFF = [640 * i for i in range(16)]


_G = 10
_NGROUP = NCHUNK // _G


def _sc_agg_body(with_counts, src_a, dst_a, tab_a, src_b, dst_b, tab_b,
                 zeros64, zeros16, ones_hbm,
                 *refs):
    n_out = 4 if with_counts else 2
    outs, refs = refs[:n_out], refs[n_out:]
    if with_counts:
        agg_a_out, agg_b_out, cnt_a_out, cnt_b_out = outs
    else:
        agg_a_out, agg_b_out = outs
        cnt_a_out = cnt_b_out = None
    isx = refs[0:_G]
    idx = refs[_G:2 * _G]
    rws = refs[2 * _G:3 * _G]
    refs = refs[3 * _G:]
    if with_counts:
        ones_v, acc, cacc, sem_i, sem_g, sem_s = refs
    else:
        acc, sem_i, sem_g, sem_s = refs
        cacc = None

    c = lax.axis_index("c")
    s = lax.axis_index("s")
    wid = s * NC + c
    base = wid * EPW

    if with_counts:
        pltpu.sync_copy(ones_hbm, ones_v)

    def run_direction(src_hbm, dst_hbm, tab_hbm, agg_out, cnt_out):
        for t in range(NS):
            @pl.when(s == t)
            def _():
                sl = pl.ds(_ZOFF[t], _ZROWS[t])
                pltpu.sync_copy(zeros64.at[sl], acc.at[sl])
                if with_counts:
                    pltpu.sync_copy(zeros16.at[sl], cacc.at[sl])
        plsc.subcore_barrier()

        def group(g, _):
            ds = []
            for k in range(_G):
                sl = pl.ds(base + (g * _G + k) * CHUNK, CHUNK)
                ds.append(pltpu.async_copy(src_hbm.at[sl], isx[k], sem_i))
                ds.append(pltpu.async_copy(dst_hbm.at[sl], idx[k], sem_i))
            for d in ds:
                d.wait()
            ds = []
            for k in range(_G):
                ds.append(pltpu.async_copy(tab_hbm.at[isx[k]], rws[k], sem_g))
            for d in ds:
                d.wait()
            ds = []
            for k in range(_G):
                ds.append(pltpu.async_copy(rws[k], acc.at[idx[k]], sem_s,
                                           add=True))
                if with_counts:
                    ds.append(pltpu.async_copy(ones_v, cacc.at[idx[k]],
                                               sem_s, add=True))
            for d in ds:
                d.wait()
            return 0

        lax.fori_loop(0, _NGROUP, group, 0)
        plsc.subcore_barrier()

        for t in range(NS):
            @pl.when(s == t)
            def _():
                sl = pl.ds(_ZOFF[t], _ZROWS[t])
                pltpu.sync_copy(acc.at[sl], agg_out.at[c, sl])
                if with_counts:
                    pltpu.sync_copy(cacc.at[sl], cnt_out.at[c, sl])
        plsc.subcore_barrier()

    run_direction(src_a, dst_a, tab_a, agg_a_out, cnt_a_out)
    run_direction(src_b, dst_b, tab_b, agg_b_out, cnt_b_out)


def _make_sc_agg(with_counts):
    mesh = plsc.VectorSubcoreMesh(core_axis_name="c", subcore_axis_name="s",
                                  num_cores=NC, num_subcores=NS)
    out_type = [
        jax.ShapeDtypeStruct((NC, N_NODES, D), jnp.float32),
        jax.ShapeDtypeStruct((NC, N_NODES, D), jnp.float32),
    ]
    if with_counts:
        out_type += [
            jax.ShapeDtypeStruct((NC, N_NODES, CW), jnp.float32),
            jax.ShapeDtypeStruct((NC, N_NODES, CW), jnp.float32),
        ]
    scratch = ([pltpu.VMEM((CHUNK,), jnp.int32)] * (2 * _G) +
               [pltpu.VMEM((CHUNK, D), jnp.float32)] * _G)
    if with_counts:
        scratch += [pltpu.VMEM((CHUNK, CW), jnp.float32)]
    scratch += [pltpu.VMEM_SHARED((N_NODES, D), jnp.float32)]
    if with_counts:
        scratch += [pltpu.VMEM_SHARED((N_NODES, CW), jnp.float32)]
    scratch += [pltpu.SemaphoreType.DMA] * 3
    return pl.kernel(
        functools.partial(_sc_agg_body, with_counts),
        out_type=tuple(out_type),
        mesh=mesh,
        scratch_types=tuple(scratch),
        compiler_params=pltpu.CompilerParams(use_tc_tiling_on_sc=False),
    )


def _proj_body(xc, wc, bc, xd, wd, bd, xs, ws, bs, oc, od, os_):
    dn = (((1,), (1,)), ((), ()))
    oc[...] = lax.dot_general(xc[...], wc[...], dn,
                              preferred_element_type=jnp.float32) + bc[...]
    od[...] = lax.dot_general(xd[...], wd[...], dn,
                              preferred_element_type=jnp.float32) + bd[...]
    os_[...] = jnp.maximum(
        lax.dot_general(xs[...], ws[...], dn,
                        preferred_element_type=jnp.float32) + bs[...], 0.0)


def _conv_body(relu, aggp_a, cntp_a, xdst_a, wl_a, bl_a, wr_a,
               aggp_b, cntp_b, xdst_b, wl_b, bl_b, wr_b, oa, ob):
    dn = (((1,), (1,)), ((), ()))

    def one(aggp, cntp, xdst, wl, bl, wr, out):
        agg = aggp[0] + aggp[1]
        cnt = cntp[0, :, 0] + cntp[1, :, 0]
        inv = 1.0 / jnp.maximum(cnt, 1.0)
        mean = agg * inv[:, None]
        r = (lax.dot_general(mean, wl[...], dn,
                             preferred_element_type=jnp.float32) + bl[...] +
             lax.dot_general(xdst[...], wr[...], dn,
                             preferred_element_type=jnp.float32))
        out[...] = jnp.maximum(r, 0.0) if relu else r

    one(aggp_a, cntp_a, xdst_a, wl_a, bl_a, wr_a, oa)
    one(aggp_b, cntp_b, xdst_b, wl_b, bl_b, wr_b, ob)


_BM = 1000
_GRID = N_NODES // _BM


def _proj_call(xc, Wc, bc, xd, Wd, bd, xs, Ws, bs):
    xspec = pl.BlockSpec((_BM, IN_DIM), lambda m: (m, 0))
    wspec = pl.BlockSpec((D, IN_DIM), lambda m: (0, 0))
    bspec = pl.BlockSpec((1, D), lambda m: (0, 0))
    ospec = pl.BlockSpec((_BM, D), lambda m: (m, 0))
    oshape = jax.ShapeDtypeStruct((N_NODES, D), jnp.float32)
    return pl.pallas_call(
        _proj_body,
        grid=(_GRID,),
        in_specs=[xspec, wspec, bspec] * 3,
        out_specs=[ospec] * 3,
        out_shape=[oshape] * 3,
    )(xc, Wc, bc.reshape(1, D), xd, Wd, bd.reshape(1, D),
      xs, Ws, bs.reshape(1, D))


def _conv_call(relu, aggp_a, cntp_a, xdst_a, wl_a, bl_a, wr_a,
               aggp_b, cntp_b, xdst_b, wl_b, bl_b, wr_b):
    aspec = pl.BlockSpec((NC, _BM, D), lambda m: (0, m, 0))
    cspec = pl.BlockSpec((NC, _BM, CW), lambda m: (0, m, 0))
    xspec = pl.BlockSpec((_BM, D), lambda m: (m, 0))
    wspec = pl.BlockSpec((D, D), lambda m: (0, 0))
    bspec = pl.BlockSpec((1, D), lambda m: (0, 0))
    ospec = pl.BlockSpec((_BM, D), lambda m: (m, 0))
    oshape = jax.ShapeDtypeStruct((N_NODES, D), jnp.float32)
    return pl.pallas_call(
        functools.partial(_conv_body, relu),
        grid=(_GRID,),
        in_specs=[aspec, cspec, xspec, wspec, bspec, wspec] * 2,
        out_specs=[ospec] * 2,
        out_shape=[oshape] * 2,
    )(aggp_a, cntp_a, xdst_a, wl_a, bl_a.reshape(1, D), wr_a,
      aggp_b, cntp_b, xdst_b, wl_b, bl_b.reshape(1, D), wr_b)


def kernel(x_chemical, x_disease, x_side_effect, edge_index_treats,
           edge_index_rev_treats,
           Wp_c, bp_c, Wp_d, bp_d, Wp_s, bp_s,
           Wl1_td, bl1_td, Wr1_td, Wl1_dc, bl1_dc, Wr1_dc,
           Wl2_td, bl2_td, Wr2_td, Wl2_dc, bl2_dc, Wr2_dc):
    src_td = edge_index_treats[0]
    dst_td = edge_index_treats[1]
    src_dc = edge_index_rev_treats[0]
    dst_dc = edge_index_rev_treats[1]
    zeros64 = jnp.zeros((N_NODES, D), jnp.float32)
    zeros16 = jnp.zeros((N_NODES, CW), jnp.float32)
    ones = jnp.ones((CHUNK, CW), jnp.float32)

    xc, xd, s1 = _proj_call(x_chemical, Wp_c, bp_c, x_disease, Wp_d, bp_d,
                            x_side_effect, Wp_s, bp_s)

    sc1 = _make_sc_agg(True)
    aggp_td, aggp_dc, cntp_td, cntp_dc = sc1(
        src_td, dst_td, xc, src_dc, dst_dc, xd, zeros64, zeros16, ones)

    d1, c1 = _conv_call(True,
                        aggp_td, cntp_td, xd, Wl1_td, bl1_td, Wr1_td,
                        aggp_dc, cntp_dc, xc, Wl1_dc, bl1_dc, Wr1_dc)

    sc2 = _make_sc_agg(False)
    aggp2_td, aggp2_dc = sc2(
        src_td, dst_td, c1, src_dc, dst_dc, d1, zeros64, zeros16, ones)

    d2, c2 = _conv_call(False,
                        aggp2_td, cntp_td, d1, Wl2_td, bl2_td, Wr2_td,
                        aggp2_dc, cntp_dc, c1, Wl2_dc, bl2_dc, Wr2_dc)

    return c2, d2, s1


# speedup vs baseline: 22.9965x; 1.1816x over previous
import functools

import jax
import jax.numpy as jnp
from jax import lax
from jax.experimental import pallas as pl
from jax.experimental.pallas import tpu as pltpu
from jax.experimental.pallas import tpu_sc as plsc

N_NODES = 10000
IN_DIM = 128
D = 64
CW = 8
E = 640000

NC = 2
NS = 16
NW = NC * NS
EPW = E // NW
CHUNK = 80
NCHUNK = EPW // CHUNK

_ZROWS = [640] * 15 + [400]
_ZOFF = [640 * i for i in range(16)]


_G = 10
_NGROUP = NCHUNK // _G


def _sc_agg_body(with_counts, src_a, dst_a, tab_a, src_b, dst_b, tab_b,
                 zeros64, zeros16, ones_hbm,
                 *refs):
    n_out = 4 if with_counts else 2
    outs, refs = refs[:n_out], refs[n_out:]
    if with_counts:
        agg_a_out, agg_b_out, cnt_a_out, cnt_b_out = outs
    else:
        agg_a_out, agg_b_out = outs
        cnt_a_out = cnt_b_out = None
    isx = refs[0:_G]
    idx = refs[_G:2 * _G]
    rws = refs[2 * _G:3 * _G]
    refs = refs[3 * _G:]
    if with_counts:
        ones_v, acc, cacc, sem_i, sem_g, sem_s = refs
    else:
        acc, sem_i, sem_g, sem_s = refs
        cacc = None

    c = lax.axis_index("c")
    s = lax.axis_index("s")
    wid = s * NC + c
    base = wid * EPW

    if with_counts:
        pltpu.sync_copy(ones_hbm, ones_v)

    def run_direction(src_hbm, dst_hbm, tab_hbm, agg_out, cnt_out):
        for t in range(NS):
            @pl.when(s == t)
            def _():
                sl = pl.ds(_ZOFF[t], _ZROWS[t])
                pltpu.sync_copy(zeros64.at[sl], acc.at[sl])
                if with_counts:
                    pltpu.sync_copy(zeros16.at[sl], cacc.at[sl])
        plsc.subcore_barrier()

        def drain_scatters():
            for k in range(_G):
                pltpu.make_async_copy(rws[k], acc.at[idx[k]], sem_s).wait()
                if with_counts:
                    pltpu.make_async_copy(ones_v, cacc.at[idx[k]],
                                          sem_s).wait()

        def group(g, _):
            @pl.when(g > 0)
            def _():
                drain_scatters()
            dsi = []
            for k in range(_G):
                sl = pl.ds(base + (g * _G + k) * CHUNK, CHUNK)
                dsi.append(pltpu.async_copy(src_hbm.at[sl], isx[k], sem_i))
                dsi.append(pltpu.async_copy(dst_hbm.at[sl], idx[k], sem_i))
            dsg = []
            for k in range(_G):
                dsi[2 * k].wait()
                dsi[2 * k + 1].wait()
                dsg.append(pltpu.async_copy(tab_hbm.at[isx[k]], rws[k],
                                            sem_g))
            for k in range(_G):
                dsg[k].wait()
                pltpu.async_copy(rws[k], acc.at[idx[k]], sem_s, add=True)
                if with_counts:
                    pltpu.async_copy(ones_v, cacc.at[idx[k]], sem_s,
                                     add=True)
            return 0

        lax.fori_loop(0, _NGROUP, group, 0)
        drain_scatters()
        plsc.subcore_barrier()

        for t in range(NS):
            @pl.when(s == t)
            def _():
                sl = pl.ds(_ZOFF[t], _ZROWS[t])
                pltpu.sync_copy(acc.at[sl], agg_out.at[c, sl])
                if with_counts:
                    pltpu.sync_copy(cacc.at[sl], cnt_out.at[c, sl])
        plsc.subcore_barrier()

    run_direction(src_a, dst_a, tab_a, agg_a_out, cnt_a_out)
    run_direction(src_b, dst_b, tab_b, agg_b_out, cnt_b_out)


def _make_sc_agg(with_counts):
    mesh = plsc.VectorSubcoreMesh(core_axis_name="c", subcore_axis_name="s",
                                  num_cores=NC, num_subcores=NS)
    out_type = [
        jax.ShapeDtypeStruct((NC, N_NODES, D), jnp.float32),
        jax.ShapeDtypeStruct((NC, N_NODES, D), jnp.float32),
    ]
    if with_counts:
        out_type += [
            jax.ShapeDtypeStruct((NC, N_NODES, CW), jnp.float32),
            jax.ShapeDtypeStruct((NC, N_NODES, CW), jnp.float32),
        ]
    scratch = ([pltpu.VMEM((CHUNK,), jnp.int32)] * (2 * _G) +
               [pltpu.VMEM((CHUNK, D), jnp.float32)] * _G)
    if with_counts:
        scratch += [pltpu.VMEM((CHUNK, CW), jnp.float32)]
    scratch += [pltpu.VMEM_SHARED((N_NODES, D), jnp.float32)]
    if with_counts:
        scratch += [pltpu.VMEM_SHARED((N_NODES, CW), jnp.float32)]
    scratch += [pltpu.SemaphoreType.DMA] * 3
    return pl.kernel(
        functools.partial(_sc_agg_body, with_counts),
        out_type=tuple(out_type),
        mesh=mesh,
        scratch_types=tuple(scratch),
        compiler_params=pltpu.CompilerParams(use_tc_tiling_on_sc=False),
    )


def _proj_body(xc, wc, bc, xd, wd, bd, xs, ws, bs, oc, od, os_):
    dn = (((1,), (1,)), ((), ()))
    oc[...] = lax.dot_general(xc[...], wc[...], dn,
                              preferred_element_type=jnp.float32) + bc[...]
    od[...] = lax.dot_general(xd[...], wd[...], dn,
                              preferred_element_type=jnp.float32) + bd[...]
    os_[...] = jnp.maximum(
        lax.dot_general(xs[...], ws[...], dn,
                        preferred_element_type=jnp.float32) + bs[...], 0.0)


def _conv_body(relu, aggp_a, cntp_a, xdst_a, wl_a, bl_a, wr_a,
               aggp_b, cntp_b, xdst_b, wl_b, bl_b, wr_b, oa, ob):
    dn = (((1,), (1,)), ((), ()))

    def one(aggp, cntp, xdst, wl, bl, wr, out):
        agg = aggp[0] + aggp[1]
        cnt = cntp[0, :, 0] + cntp[1, :, 0]
        inv = 1.0 / jnp.maximum(cnt, 1.0)
        mean = agg * inv[:, None]
        r = (lax.dot_general(mean, wl[...], dn,
                             preferred_element_type=jnp.float32) + bl[...] +
             lax.dot_general(xdst[...], wr[...], dn,
                             preferred_element_type=jnp.float32))
        out[...] = jnp.maximum(r, 0.0) if relu else r

    one(aggp_a, cntp_a, xdst_a, wl_a, bl_a, wr_a, oa)
    one(aggp_b, cntp_b, xdst_b, wl_b, bl_b, wr_b, ob)


_BM = 1000
_GRID = N_NODES // _BM


def _proj_call(xc, Wc, bc, xd, Wd, bd, xs, Ws, bs):
    xspec = pl.BlockSpec((_BM, IN_DIM), lambda m: (m, 0))
    wspec = pl.BlockSpec((D, IN_DIM), lambda m: (0, 0))
    bspec = pl.BlockSpec((1, D), lambda m: (0, 0))
    ospec = pl.BlockSpec((_BM, D), lambda m: (m, 0))
    oshape = jax.ShapeDtypeStruct((N_NODES, D), jnp.float32)
    return pl.pallas_call(
        _proj_body,
        grid=(_GRID,),
        in_specs=[xspec, wspec, bspec] * 3,
        out_specs=[ospec] * 3,
        out_shape=[oshape] * 3,
    )(xc, Wc, bc.reshape(1, D), xd, Wd, bd.reshape(1, D),
      xs, Ws, bs.reshape(1, D))


def _conv_call(relu, aggp_a, cntp_a, xdst_a, wl_a, bl_a, wr_a,
               aggp_b, cntp_b, xdst_b, wl_b, bl_b, wr_b):
    aspec = pl.BlockSpec((NC, _BM, D), lambda m: (0, m, 0))
    cspec = pl.BlockSpec((NC, _BM, CW), lambda m: (0, m, 0))
    xspec = pl.BlockSpec((_BM, D), lambda m: (m, 0))
    wspec = pl.BlockSpec((D, D), lambda m: (0, 0))
    bspec = pl.BlockSpec((1, D), lambda m: (0, 0))
    ospec = pl.BlockSpec((_BM, D), lambda m: (m, 0))
    oshape = jax.ShapeDtypeStruct((N_NODES, D), jnp.float32)
    return pl.pallas_call(
        functools.partial(_conv_body, relu),
        grid=(_GRID,),
        in_specs=[aspec, cspec, xspec, wspec, bspec, wspec] * 2,
        out_specs=[ospec] * 2,
        out_shape=[oshape] * 2,
    )(aggp_a, cntp_a, xdst_a, wl_a, bl_a.reshape(1, D), wr_a,
      aggp_b, cntp_b, xdst_b, wl_b, bl_b.reshape(1, D), wr_b)


def kernel(x_chemical, x_disease, x_side_effect, edge_index_treats,
           edge_index_rev_treats,
           Wp_c, bp_c, Wp_d, bp_d, Wp_s, bp_s,
           Wl1_td, bl1_td, Wr1_td, Wl1_dc, bl1_dc, Wr1_dc,
           Wl2_td, bl2_td, Wr2_td, Wl2_dc, bl2_dc, Wr2_dc):
    src_td = edge_index_treats[0]
    dst_td = edge_index_treats[1]
    src_dc = edge_index_rev_treats[0]
    dst_dc = edge_index_rev_treats[1]
    zeros64 = jnp.zeros((N_NODES, D), jnp.float32)
    zeros16 = jnp.zeros((N_NODES, CW), jnp.float32)
    ones = jnp.ones((CHUNK, CW), jnp.float32)

    xc, xd, s1 = _proj_call(x_chemical, Wp_c, bp_c, x_disease, Wp_d, bp_d,
                            x_side_effect, Wp_s, bp_s)

    sc1 = _make_sc_agg(True)
    aggp_td, aggp_dc, cntp_td, cntp_dc = sc1(
        src_td, dst_td, xc, src_dc, dst_dc, xd, zeros64, zeros16, ones)

    d1, c1 = _conv_call(True,
                        aggp_td, cntp_td, xd, Wl1_td, bl1_td, Wr1_td,
                        aggp_dc, cntp_dc, xc, Wl1_dc, bl1_dc, Wr1_dc)

    sc2 = _make_sc_agg(False)
    aggp2_td, aggp2_dc = sc2(
        src_td, dst_td, c1, src_dc, dst_dc, d1, zeros64, zeros16, ones)

    d2, c2 = _conv_call(False,
                        aggp2_td, cntp_td, d1, Wl2_td, bl2_td, Wr2_td,
                        aggp2_dc, cntp_dc, c1, Wl2_dc, bl2_dc, Wr2_dc)

    return c2, d2, s1


# speedup vs baseline: 23.4024x; 1.0177x over previous
import functools

import jax
import jax.numpy as jnp
from jax import lax
from jax.experimental import pallas as pl
from jax.experimental.pallas import tpu as pltpu
from jax.experimental.pallas import tpu_sc as plsc

N_NODES = 10000
IN_DIM = 128
D = 64
CW = 8
E = 640000

NC = 2
NS = 16
NW = NC * NS
EPW = E // NW
CHUNK = 80
NCHUNK = EPW // CHUNK
_G = 10
_NGROUP = NCHUNK // _G

_ZROWS = [640] * 15 + [400]
_ZOFF = [640 * i for i in range(16)]

_SC_PARAMS = pltpu.CompilerParams(use_tc_tiling_on_sc=False)
_MESH = dict(core_axis_name="c", subcore_axis_name="s",
             num_cores=NC, num_subcores=NS)


def _sc_agg_body(src_a, dst_a, tab_a, src_b, dst_b, tab_b, zeros64, *refs):
    agg_a_out, agg_b_out = refs[:2]
    refs = refs[2:]
    isx = refs[0:_G]
    idx = refs[_G:2 * _G]
    rws = refs[2 * _G:3 * _G]
    acc, sem_i, sem_g, sem_s = refs[3 * _G:]

    c = lax.axis_index("c")
    s = lax.axis_index("s")
    wid = s * NC + c
    base = wid * EPW

    def run_direction(src_hbm, dst_hbm, tab_hbm, agg_out):
        for t in range(NS):
            @pl.when(s == t)
            def _():
                sl = pl.ds(_ZOFF[t], _ZROWS[t])
                pltpu.sync_copy(zeros64.at[sl], acc.at[sl])
        plsc.subcore_barrier()

        def drain_scatters():
            for k in range(_G):
                pltpu.make_async_copy(rws[k], acc.at[idx[k]], sem_s).wait()

        def group(g, _):
            @pl.when(g > 0)
            def _():
                drain_scatters()
            dsi = []
            for k in range(_G):
                sl = pl.ds(base + (g * _G + k) * CHUNK, CHUNK)
                dsi.append(pltpu.async_copy(src_hbm.at[sl], isx[k], sem_i))
                dsi.append(pltpu.async_copy(dst_hbm.at[sl], idx[k], sem_i))
            dsg = []
            for k in range(_G):
                dsi[2 * k].wait()
                dsi[2 * k + 1].wait()
                dsg.append(pltpu.async_copy(tab_hbm.at[isx[k]], rws[k],
                                            sem_g))
            for k in range(_G):
                dsg[k].wait()
                pltpu.async_copy(rws[k], acc.at[idx[k]], sem_s, add=True)
            return 0

        lax.fori_loop(0, _NGROUP, group, 0)
        drain_scatters()
        plsc.subcore_barrier()

        for t in range(NS):
            @pl.when(s == t)
            def _():
                sl = pl.ds(_ZOFF[t], _ZROWS[t])
                pltpu.sync_copy(acc.at[sl], agg_out.at[c, sl])
        plsc.subcore_barrier()

    run_direction(src_a, dst_a, tab_a, agg_a_out)
    run_direction(src_b, dst_b, tab_b, agg_b_out)


def _make_sc_agg():
    mesh = plsc.VectorSubcoreMesh(**_MESH)
    out_type = (
        jax.ShapeDtypeStruct((NC, N_NODES, D), jnp.float32),
        jax.ShapeDtypeStruct((NC, N_NODES, D), jnp.float32),
    )
    scratch = ([pltpu.VMEM((CHUNK,), jnp.int32)] * (2 * _G) +
               [pltpu.VMEM((CHUNK, D), jnp.float32)] * _G +
               [pltpu.VMEM_SHARED((N_NODES, D), jnp.float32)] +
               [pltpu.SemaphoreType.DMA] * 3)
    return pl.kernel(
        _sc_agg_body,
        out_type=out_type,
        mesh=mesh,
        scratch_types=tuple(scratch),
        compiler_params=_SC_PARAMS,
    )


def _sc_cnt_body(dst_a, dst_b, zeros_cw, ones_hbm, *refs):
    cnt_a_out, cnt_b_out = refs[:2]
    refs = refs[2:]
    ida = refs[0:_G]
    idb = refs[_G:2 * _G]
    ones_v, cacc_a, cacc_b, sem_i, sem_s = refs[2 * _G:]

    c = lax.axis_index("c")
    s = lax.axis_index("s")
    wid = s * NC + c
    base = wid * EPW

    for t in range(NS):
        @pl.when(s == t)
        def _():
            sl = pl.ds(_ZOFF[t], _ZROWS[t])
            pltpu.sync_copy(zeros_cw.at[sl], cacc_a.at[sl])
            pltpu.sync_copy(zeros_cw.at[sl], cacc_b.at[sl])
    pltpu.sync_copy(ones_hbm, ones_v)
    plsc.subcore_barrier()

    def drain_scatters():
        for k in range(_G):
            pltpu.make_async_copy(ones_v, cacc_a.at[ida[k]], sem_s).wait()
            pltpu.make_async_copy(ones_v, cacc_b.at[idb[k]], sem_s).wait()

    def group(g, _):
        @pl.when(g > 0)
        def _():
            drain_scatters()
        dsi = []
        for k in range(_G):
            sl = pl.ds(base + (g * _G + k) * CHUNK, CHUNK)
            dsi.append(pltpu.async_copy(dst_a.at[sl], ida[k], sem_i))
            dsi.append(pltpu.async_copy(dst_b.at[sl], idb[k], sem_i))
        for k in range(_G):
            dsi[2 * k].wait()
            pltpu.async_copy(ones_v, cacc_a.at[ida[k]], sem_s, add=True)
            dsi[2 * k + 1].wait()
            pltpu.async_copy(ones_v, cacc_b.at[idb[k]], sem_s, add=True)
        return 0

    lax.fori_loop(0, _NGROUP, group, 0)
    drain_scatters()
    plsc.subcore_barrier()

    for t in range(NS):
        @pl.when(s == t)
        def _():
            sl = pl.ds(_ZOFF[t], _ZROWS[t])
            pltpu.sync_copy(cacc_a.at[sl], cnt_a_out.at[c, sl])
            pltpu.sync_copy(cacc_b.at[sl], cnt_b_out.at[c, sl])


def _make_sc_cnt():
    mesh = plsc.VectorSubcoreMesh(**_MESH)
    out_type = (
        jax.ShapeDtypeStruct((NC, N_NODES, CW), jnp.float32),
        jax.ShapeDtypeStruct((NC, N_NODES, CW), jnp.float32),
    )
    scratch = ([pltpu.VMEM((CHUNK,), jnp.int32)] * (2 * _G) +
               [pltpu.VMEM((CHUNK, CW), jnp.float32)] +
               [pltpu.VMEM_SHARED((N_NODES, CW), jnp.float32)] * 2 +
               [pltpu.SemaphoreType.DMA] * 2)
    return pl.kernel(
        _sc_cnt_body,
        out_type=out_type,
        mesh=mesh,
        scratch_types=tuple(scratch),
        compiler_params=_SC_PARAMS,
    )


def _proj_body(xc, wc, bc, xd, wd, bd, xs, ws, bs, oc, od, os_):
    dn = (((1,), (1,)), ((), ()))
    oc[...] = lax.dot_general(xc[...], wc[...], dn,
                              preferred_element_type=jnp.float32) + bc[...]
    od[...] = lax.dot_general(xd[...], wd[...], dn,
                              preferred_element_type=jnp.float32) + bd[...]
    os_[...] = jnp.maximum(
        lax.dot_general(xs[...], ws[...], dn,
                        preferred_element_type=jnp.float32) + bs[...], 0.0)


def _conv_body(relu, aggp_a, cntp_a, xdst_a, wl_a, bl_a, wr_a,
               aggp_b, cntp_b, xdst_b, wl_b, bl_b, wr_b, oa, ob):
    dn = (((1,), (1,)), ((), ()))

    def one(aggp, cntp, xdst, wl, bl, wr, out):
        agg = aggp[0] + aggp[1]
        cnt = cntp[0, :, 0] + cntp[1, :, 0]
        inv = 1.0 / jnp.maximum(cnt, 1.0)
        mean = agg * inv[:, None]
        r = (lax.dot_general(mean, wl[...], dn,
                             preferred_element_type=jnp.float32) + bl[...] +
             lax.dot_general(xdst[...], wr[...], dn,
                             preferred_element_type=jnp.float32))
        out[...] = jnp.maximum(r, 0.0) if relu else r

    one(aggp_a, cntp_a, xdst_a, wl_a, bl_a, wr_a, oa)
    one(aggp_b, cntp_b, xdst_b, wl_b, bl_b, wr_b, ob)


_BM = 1000
_GRID = N_NODES // _BM


def _proj_call(xc, Wc, bc, xd, Wd, bd, xs, Ws, bs):
    xspec = pl.BlockSpec((_BM, IN_DIM), lambda m: (m, 0))
    wspec = pl.BlockSpec((D, IN_DIM), lambda m: (0, 0))
    bspec = pl.BlockSpec((1, D), lambda m: (0, 0))
    ospec = pl.BlockSpec((_BM, D), lambda m: (m, 0))
    oshape = jax.ShapeDtypeStruct((N_NODES, D), jnp.float32)
    return pl.pallas_call(
        _proj_body,
        grid=(_GRID,),
        in_specs=[xspec, wspec, bspec] * 3,
        out_specs=[ospec] * 3,
        out_shape=[oshape] * 3,
    )(xc, Wc, bc.reshape(1, D), xd, Wd, bd.reshape(1, D),
      xs, Ws, bs.reshape(1, D))


def _conv_call(relu, aggp_a, cntp_a, xdst_a, wl_a, bl_a, wr_a,
               aggp_b, cntp_b, xdst_b, wl_b, bl_b, wr_b):
    aspec = pl.BlockSpec((NC, _BM, D), lambda m: (0, m, 0))
    cspec = pl.BlockSpec((NC, _BM, CW), lambda m: (0, m, 0))
    xspec = pl.BlockSpec((_BM, D), lambda m: (m, 0))
    wspec = pl.BlockSpec((D, D), lambda m: (0, 0))
    bspec = pl.BlockSpec((1, D), lambda m: (0, 0))
    ospec = pl.BlockSpec((_BM, D), lambda m: (m, 0))
    oshape = jax.ShapeDtypeStruct((N_NODES, D), jnp.float32)
    return pl.pallas_call(
        functools.partial(_conv_body, relu),
        grid=(_GRID,),
        in_specs=[aspec, cspec, xspec, wspec, bspec, wspec] * 2,
        out_specs=[ospec] * 2,
        out_shape=[oshape] * 2,
    )(aggp_a, cntp_a, xdst_a, wl_a, bl_a.reshape(1, D), wr_a,
      aggp_b, cntp_b, xdst_b, wl_b, bl_b.reshape(1, D), wr_b)


def kernel(x_chemical, x_disease, x_side_effect, edge_index_treats,
           edge_index_rev_treats,
           Wp_c, bp_c, Wp_d, bp_d, Wp_s, bp_s,
           Wl1_td, bl1_td, Wr1_td, Wl1_dc, bl1_dc, Wr1_dc,
           Wl2_td, bl2_td, Wr2_td, Wl2_dc, bl2_dc, Wr2_dc):
    src_td = edge_index_treats[0]
    dst_td = edge_index_treats[1]
    src_dc = edge_index_rev_treats[0]
    dst_dc = edge_index_rev_treats[1]
    zeros64 = jnp.zeros((N_NODES, D), jnp.float32)
    zeros_cw = jnp.zeros((N_NODES, CW), jnp.float32)
    ones = jnp.ones((CHUNK, CW), jnp.float32)

    cntp_td, cntp_dc = _make_sc_cnt()(dst_td, dst_dc, zeros_cw, ones)

    xc, xd, s1 = _proj_call(x_chemical, Wp_c, bp_c, x_disease, Wp_d, bp_d,
                            x_side_effect, Wp_s, bp_s)

    sc_agg = _make_sc_agg()
    aggp_td, aggp_dc = sc_agg(src_td, dst_td, xc, src_dc, dst_dc, xd,
                              zeros64)

    d1, c1 = _conv_call(True,
                        aggp_td, cntp_td, xd, Wl1_td, bl1_td, Wr1_td,
                        aggp_dc, cntp_dc, xc, Wl1_dc, bl1_dc, Wr1_dc)

    aggp2_td, aggp2_dc = sc_agg(src_td, dst_td, c1, src_dc, dst_dc, d1,
                                zeros64)

    d2, c2 = _conv_call(False,
                        aggp2_td, cntp_td, d1, Wl2_td, bl2_td, Wr2_td,
                        aggp2_dc, cntp_dc, c1, Wl2_dc, bl2_dc, Wr2_dc)

    return c2, d2, s1


# speedup vs baseline: 24.3954x; 1.0424x over previous
import functools

import jax
import jax.numpy as jnp
from jax import lax
from jax.experimental import pallas as pl
from jax.experimental.pallas import tpu as pltpu
from jax.experimental.pallas import tpu_sc as plsc

N_NODES = 10000
IN_DIM = 128
D = 64
CW = 8
E = 640000

NC = 2
NS = 16
EPW = E // NS
CHUNK = 80
NCHUNK = EPW // CHUNK
_G = 10
_NGROUP = NCHUNK // _G

_ZROWS = [640] * 15 + [400]
_ZOFF = [640 * i for i in range(16)]

_SC_PARAMS = pltpu.CompilerParams(use_tc_tiling_on_sc=False)
_MESH = dict(core_axis_name="c", subcore_axis_name="s",
             num_cores=NC, num_subcores=NS)


def _sc_agg_body(src_a, dst_a, tab_a, src_b, dst_b, tab_b, zeros64, *refs):
    agg_a_out, agg_b_out = refs[:2]
    refs = refs[2:]
    isx = refs[0:_G]
    idx = refs[_G:2 * _G]
    rws = refs[2 * _G:3 * _G]
    acc, sem_i, sem_g, sem_s = refs[3 * _G:]

    c = lax.axis_index("c")
    s = lax.axis_index("s")
    base = s * EPW

    def run_direction(src_hbm, dst_hbm, tab_hbm, agg_out):
        for t in range(NS):
            @pl.when(s == t)
            def _():
                sl = pl.ds(_ZOFF[t], _ZROWS[t])
                pltpu.sync_copy(zeros64.at[sl], acc.at[sl])
        plsc.subcore_barrier()

        def drain_scatters():
            for k in range(_G):
                pltpu.make_async_copy(rws[k], acc.at[idx[k]], sem_s).wait()

        def group(g, _):
            @pl.when(g > 0)
            def _():
                drain_scatters()
            dsi = []
            for k in range(_G):
                sl = pl.ds(base + (g * _G + k) * CHUNK, CHUNK)
                dsi.append(pltpu.async_copy(src_hbm.at[sl], isx[k], sem_i))
                dsi.append(pltpu.async_copy(dst_hbm.at[sl], idx[k], sem_i))
            dsg = []
            for k in range(_G):
                dsi[2 * k].wait()
                dsi[2 * k + 1].wait()
                dsg.append(pltpu.async_copy(tab_hbm.at[isx[k]], rws[k],
                                            sem_g))
            for k in range(_G):
                dsg[k].wait()
                pltpu.async_copy(rws[k], acc.at[idx[k]], sem_s, add=True)
            return 0

        lax.fori_loop(0, _NGROUP, group, 0)
        drain_scatters()
        plsc.subcore_barrier()

        for t in range(NS):
            @pl.when(s == t)
            def _():
                sl = pl.ds(_ZOFF[t], _ZROWS[t])
                pltpu.sync_copy(acc.at[sl], agg_out.at[sl])

    @pl.when(c == 0)
    def _():
        run_direction(src_a, dst_a, tab_a, agg_a_out)

    @pl.when(c == 1)
    def _():
        run_direction(src_b, dst_b, tab_b, agg_b_out)


def _make_sc_agg():
    mesh = plsc.VectorSubcoreMesh(**_MESH)
    out_type = (
        jax.ShapeDtypeStruct((N_NODES, D), jnp.float32),
        jax.ShapeDtypeStruct((N_NODES, D), jnp.float32),
    )
    scratch = ([pltpu.VMEM((CHUNK,), jnp.int32)] * (2 * _G) +
               [pltpu.VMEM((CHUNK, D), jnp.float32)] * _G +
               [pltpu.VMEM_SHARED((N_NODES, D), jnp.float32)] +
               [pltpu.SemaphoreType.DMA] * 3)
    return pl.kernel(
        _sc_agg_body,
        out_type=out_type,
        mesh=mesh,
        scratch_types=tuple(scratch),
        compiler_params=_SC_PARAMS,
    )


def _sc_cnt_body(dst_a, dst_b, zeros_cw, ones_hbm, *refs):
    cnt_a_out, cnt_b_out = refs[:2]
    refs = refs[2:]
    idx = refs[0:_G]
    ones_v, cacc, sem_i, sem_s = refs[_G:]

    c = lax.axis_index("c")
    s = lax.axis_index("s")
    base = s * EPW

    def run_direction(dst_hbm, cnt_out):
        for t in range(NS):
            @pl.when(s == t)
            def _():
                sl = pl.ds(_ZOFF[t], _ZROWS[t])
                pltpu.sync_copy(zeros_cw.at[sl], cacc.at[sl])
        pltpu.sync_copy(ones_hbm, ones_v)
        plsc.subcore_barrier()

        def drain_scatters():
            for k in range(_G):
                pltpu.make_async_copy(ones_v, cacc.at[idx[k]], sem_s).wait()

        def group(g, _):
            @pl.when(g > 0)
            def _():
                drain_scatters()
            dsi = []
            for k in range(_G):
                sl = pl.ds(base + (g * _G + k) * CHUNK, CHUNK)
                dsi.append(pltpu.async_copy(dst_hbm.at[sl], idx[k], sem_i))
            for k in range(_G):
                dsi[k].wait()
                pltpu.async_copy(ones_v, cacc.at[idx[k]], sem_s, add=True)
            return 0

        lax.fori_loop(0, _NGROUP, group, 0)
        drain_scatters()
        plsc.subcore_barrier()

        for t in range(NS):
            @pl.when(s == t)
            def _():
                sl = pl.ds(_ZOFF[t], _ZROWS[t])
                pltpu.sync_copy(cacc.at[sl], cnt_out.at[sl])

    @pl.when(c == 0)
    def _():
        run_direction(dst_a, cnt_a_out)

    @pl.when(c == 1)
    def _():
        run_direction(dst_b, cnt_b_out)


def _make_sc_cnt():
    mesh = plsc.VectorSubcoreMesh(**_MESH)
    out_type = (
        jax.ShapeDtypeStruct((N_NODES, CW), jnp.float32),
        jax.ShapeDtypeStruct((N_NODES, CW), jnp.float32),
    )
    scratch = ([pltpu.VMEM((CHUNK,), jnp.int32)] * _G +
               [pltpu.VMEM((CHUNK, CW), jnp.float32)] +
               [pltpu.VMEM_SHARED((N_NODES, CW), jnp.float32)] +
               [pltpu.SemaphoreType.DMA] * 2)
    return pl.kernel(
        _sc_cnt_body,
        out_type=out_type,
        mesh=mesh,
        scratch_types=tuple(scratch),
        compiler_params=_SC_PARAMS,
    )


def _proj_body(xc, wc, bc, xd, wd, bd, xs, ws, bs, oc, od, os_):
    dn = (((1,), (1,)), ((), ()))
    oc[...] = lax.dot_general(xc[...], wc[...], dn,
                              preferred_element_type=jnp.float32) + bc[...]
    od[...] = lax.dot_general(xd[...], wd[...], dn,
                              preferred_element_type=jnp.float32) + bd[...]
    os_[...] = jnp.maximum(
        lax.dot_general(xs[...], ws[...], dn,
                        preferred_element_type=jnp.float32) + bs[...], 0.0)


def _conv_body(relu, agg_a, cnt_a, xdst_a, wl_a, bl_a, wr_a,
               agg_b, cnt_b, xdst_b, wl_b, bl_b, wr_b, oa, ob):
    dn = (((1,), (1,)), ((), ()))

    def one(agg, cnt, xdst, wl, bl, wr, out):
        inv = 1.0 / jnp.maximum(cnt[:, 0], 1.0)
        mean = agg[...] * inv[:, None]
        r = (lax.dot_general(mean, wl[...], dn,
                             preferred_element_type=jnp.float32) + bl[...] +
             lax.dot_general(xdst[...], wr[...], dn,
                             preferred_element_type=jnp.float32))
        out[...] = jnp.maximum(r, 0.0) if relu else r

    one(agg_a, cnt_a, xdst_a, wl_a, bl_a, wr_a, oa)
    one(agg_b, cnt_b, xdst_b, wl_b, bl_b, wr_b, ob)


_BM = 1000
_GRID = N_NODES // _BM


def _proj_call(xc, Wc, bc, xd, Wd, bd, xs, Ws, bs):
    xspec = pl.BlockSpec((_BM, IN_DIM), lambda m: (m, 0))
    wspec = pl.BlockSpec((D, IN_DIM), lambda m: (0, 0))
    bspec = pl.BlockSpec((1, D), lambda m: (0, 0))
    ospec = pl.BlockSpec((_BM, D), lambda m: (m, 0))
    oshape = jax.ShapeDtypeStruct((N_NODES, D), jnp.float32)
    return pl.pallas_call(
        _proj_body,
        grid=(_GRID,),
        in_specs=[xspec, wspec, bspec] * 3,
        out_specs=[ospec] * 3,
        out_shape=[oshape] * 3,
    )(xc, Wc, bc.reshape(1, D), xd, Wd, bd.reshape(1, D),
      xs, Ws, bs.reshape(1, D))


def _conv_call(relu, agg_a, cnt_a, xdst_a, wl_a, bl_a, wr_a,
               agg_b, cnt_b, xdst_b, wl_b, bl_b, wr_b):
    aspec = pl.BlockSpec((_BM, D), lambda m: (m, 0))
    cspec = pl.BlockSpec((_BM, CW), lambda m: (m, 0))
    wspec = pl.BlockSpec((D, D), lambda m: (0, 0))
    bspec = pl.BlockSpec((1, D), lambda m: (0, 0))
    oshape = jax.ShapeDtypeStruct((N_NODES, D), jnp.float32)
    return pl.pallas_call(
        functools.partial(_conv_body, relu),
        grid=(_GRID,),
        in_specs=[aspec, cspec, aspec, wspec, bspec, wspec] * 2,
        out_specs=[aspec] * 2,
        out_shape=[oshape] * 2,
    )(agg_a, cnt_a, xdst_a, wl_a, bl_a.reshape(1, D), wr_a,
      agg_b, cnt_b, xdst_b, wl_b, bl_b.reshape(1, D), wr_b)


def kernel(x_chemical, x_disease, x_side_effect, edge_index_treats,
           edge_index_rev_treats,
           Wp_c, bp_c, Wp_d, bp_d, Wp_s, bp_s,
           Wl1_td, bl1_td, Wr1_td, Wl1_dc, bl1_dc, Wr1_dc,
           Wl2_td, bl2_td, Wr2_td, Wl2_dc, bl2_dc, Wr2_dc):
    src_td = edge_index_treats[0]
    dst_td = edge_index_treats[1]
    src_dc = edge_index_rev_treats[0]
    dst_dc = edge_index_rev_treats[1]
    zeros64 = jnp.zeros((N_NODES, D), jnp.float32)
    zeros_cw = jnp.zeros((N_NODES, CW), jnp.float32)
    ones = jnp.ones((CHUNK, CW), jnp.float32)

    cnt_td, cnt_dc = _make_sc_cnt()(dst_td, dst_dc, zeros_cw, ones)

    xc, xd, s1 = _proj_call(x_chemical, Wp_c, bp_c, x_disease, Wp_d, bp_d,
                            x_side_effect, Wp_s, bp_s)

    sc_agg = _make_sc_agg()
    agg_td, agg_dc = sc_agg(src_td, dst_td, xc, src_dc, dst_dc, xd, zeros64)

    d1, c1 = _conv_call(True,
                        agg_td, cnt_td, xd, Wl1_td, bl1_td, Wr1_td,
                        agg_dc, cnt_dc, xc, Wl1_dc, bl1_dc, Wr1_dc)

    agg2_td, agg2_dc = sc_agg(src_td, dst_td, c1, src_dc, dst_dc, d1,
                              zeros64)

    d2, c2 = _conv_call(False,
                        agg2_td, cnt_td, d1, Wl2_td, bl2_td, Wr2_td,
                        agg2_dc, cnt_dc, c1, Wl2_dc, bl2_dc, Wr2_dc)

    return c2, d2, s1


# speedup vs baseline: 24.9957x; 1.0246x over previous
import functools

import jax
import jax.numpy as jnp
from jax import lax
from jax.experimental import pallas as pl
from jax.experimental.pallas import tpu as pltpu
from jax.experimental.pallas import tpu_sc as plsc

N_NODES = 10000
IN_DIM = 128
D = 64
CW = 8
E = 640000

NC = 2
NS = 16
EPW = E // NS
CHUNK = 128
_G = 8
_NGROUP = 39
TAIL = EPW - _NGROUP * _G * CHUNK

_ZROWS = [640] * 15 + [400]
_ZOFF = [640 * i for i in range(16)]

_SC_PARAMS = pltpu.CompilerParams(use_tc_tiling_on_sc=False)
_MESH = dict(core_axis_name="c", subcore_axis_name="s",
             num_cores=NC, num_subcores=NS)


def _sc_agg_body(with_counts, src_a, dst_a, tab_a, src_b, dst_b, tab_b,
                 zeros64, zeros_cw, ones_hbm, *refs):
    if with_counts:
        agg_a_out, agg_b_out, cnt_a_out, cnt_b_out = refs[:4]
        refs = refs[4:]
    else:
        agg_a_out, agg_b_out = refs[:2]
        cnt_a_out = cnt_b_out = None
        refs = refs[2:]
    isx = refs[0:_G]
    idx = refs[_G:2 * _G]
    rws = refs[2 * _G:3 * _G]
    isx_t, idx_t, rws_t = refs[3 * _G:3 * _G + 3]
    refs = refs[3 * _G + 3:]
    if with_counts:
        ones_v, acc, cacc, sem_i, sem_g, sem_s = refs
    else:
        acc, sem_i, sem_g, sem_s = refs
        cacc = None

    c = lax.axis_index("c")
    s = lax.axis_index("s")
    base = s * EPW

    def run_direction(src_hbm, dst_hbm, tab_hbm, agg_out, cnt_out):
        for t in range(NS):
            @pl.when(s == t)
            def _():
                sl = pl.ds(_ZOFF[t], _ZROWS[t])
                pltpu.sync_copy(zeros64.at[sl], acc.at[sl])
                if with_counts:
                    pltpu.sync_copy(zeros_cw.at[sl], cacc.at[sl])
        if with_counts:
            pltpu.sync_copy(ones_hbm, ones_v)
        plsc.subcore_barrier()

        def drain_scatters():
            for k in range(_G):
                pltpu.make_async_copy(rws[k], acc.at[idx[k]], sem_s).wait()
                if with_counts:
                    pltpu.make_async_copy(ones_v, cacc.at[idx[k]],
                                          sem_s).wait()

        def group(g, _):
            @pl.when(g > 0)
            def _():
                drain_scatters()
            dsi = []
            for k in range(_G):
                sl = pl.ds(base + (g * _G + k) * CHUNK, CHUNK)
                dsi.append(pltpu.async_copy(src_hbm.at[sl], isx[k], sem_i))
                dsi.append(pltpu.async_copy(dst_hbm.at[sl], idx[k], sem_i))
            dsg = []
            for k in range(_G):
                dsi[2 * k].wait()
                dsi[2 * k + 1].wait()
                dsg.append(pltpu.async_copy(tab_hbm.at[isx[k]], rws[k],
                                            sem_g))
            for k in range(_G):
                dsg[k].wait()
                pltpu.async_copy(rws[k], acc.at[idx[k]], sem_s, add=True)
                if with_counts:
                    pltpu.async_copy(ones_v, cacc.at[idx[k]], sem_s,
                                     add=True)
            return 0

        lax.fori_loop(0, _NGROUP, group, 0)
        drain_scatters()

        sl = pl.ds(base + _NGROUP * _G * CHUNK, TAIL)
        pltpu.sync_copy(src_hbm.at[sl], isx_t)
        pltpu.sync_copy(dst_hbm.at[sl], idx_t)
        pltpu.async_copy(tab_hbm.at[isx_t], rws_t, sem_g).wait()
        pltpu.async_copy(rws_t, acc.at[idx_t], sem_s, add=True)
        if with_counts:
            pltpu.async_copy(ones_v.at[pl.ds(0, TAIL)], cacc.at[idx_t],
                             sem_s, add=True)
            pltpu.make_async_copy(ones_v.at[pl.ds(0, TAIL)],
                                  cacc.at[idx_t], sem_s).wait()
        pltpu.make_async_copy(rws_t, acc.at[idx_t], sem_s).wait()
        plsc.subcore_barrier()

        for t in range(NS):
            @pl.when(s == t)
            def _():
                sl = pl.ds(_ZOFF[t], _ZROWS[t])
                pltpu.sync_copy(acc.at[sl], agg_out.at[sl])
                if with_counts:
                    pltpu.sync_copy(cacc.at[sl], cnt_out.at[sl])

    @pl.when(c == 0)
    def _():
        run_direction(src_a, dst_a, tab_a, agg_a_out, cnt_a_out)

    @pl.when(c == 1)
    def _():
        run_direction(src_b, dst_b, tab_b, agg_b_out, cnt_b_out)


def _make_sc_agg(with_counts):
    mesh = plsc.VectorSubcoreMesh(**_MESH)
    out_type = [
        jax.ShapeDtypeStruct((N_NODES, D), jnp.float32),
        jax.ShapeDtypeStruct((N_NODES, D), jnp.float32),
    ]
    if with_counts:
        out_type += [
            jax.ShapeDtypeStruct((N_NODES, CW), jnp.float32),
            jax.ShapeDtypeStruct((N_NODES, CW), jnp.float32),
        ]
    scratch = ([pltpu.VMEM((CHUNK,), jnp.int32)] * (2 * _G) +
               [pltpu.VMEM((CHUNK, D), jnp.float32)] * _G +
               [pltpu.VMEM((TAIL,), jnp.int32)] * 2 +
               [pltpu.VMEM((TAIL, D), jnp.float32)])
    if with_counts:
        scratch += [pltpu.VMEM((CHUNK, CW), jnp.float32)]
    scratch += [pltpu.VMEM_SHARED((N_NODES, D), jnp.float32)]
    if with_counts:
        scratch += [pltpu.VMEM_SHARED((N_NODES, CW), jnp.float32)]
    scratch += [pltpu.SemaphoreType.DMA] * 3
    return pl.kernel(
        functools.partial(_sc_agg_body, with_counts),
        out_type=tuple(out_type),
        mesh=mesh,
        scratch_types=tuple(scratch),
        compiler_params=_SC_PARAMS,
    )


def _proj_body(xc, wc, bc, xd, wd, bd, xs, ws, bs, oc, od, os_):
    dn = (((1,), (1,)), ((), ()))
    oc[...] = lax.dot_general(xc[...], wc[...], dn,
                              preferred_element_type=jnp.float32) + bc[...]
    od[...] = lax.dot_general(xd[...], wd[...], dn,
                              preferred_element_type=jnp.float32) + bd[...]
    os_[...] = jnp.maximum(
        lax.dot_general(xs[...], ws[...], dn,
                        preferred_element_type=jnp.float32) + bs[...], 0.0)


def _conv_body(relu, agg_a, cnt_a, xdst_a, wl_a, bl_a, wr_a,
               agg_b, cnt_b, xdst_b, wl_b, bl_b, wr_b, oa, ob):
    dn = (((1,), (1,)), ((), ()))

    def one(agg, cnt, xdst, wl, bl, wr, out):
        inv = 1.0 / jnp.maximum(cnt[:, 0], 1.0)
        mean = agg[...] * inv[:, None]
        r = (lax.dot_general(mean, wl[...], dn,
                             preferred_element_type=jnp.float32) + bl[...] +
             lax.dot_general(xdst[...], wr[...], dn,
                             preferred_element_type=jnp.float32))
        out[...] = jnp.maximum(r, 0.0) if relu else r

    one(agg_a, cnt_a, xdst_a, wl_a, bl_a, wr_a, oa)
    one(agg_b, cnt_b, xdst_b, wl_b, bl_b, wr_b, ob)


_BM = 1000
_GRID = N_NODES // _BM


def _proj_call(xc, Wc, bc, xd, Wd, bd, xs, Ws, bs):
    xspec = pl.BlockSpec((_BM, IN_DIM), lambda m: (m, 0))
    wspec = pl.BlockSpec((D, IN_DIM), lambda m: (0, 0))
    bspec = pl.BlockSpec((1, D), lambda m: (0, 0))
    ospec = pl.BlockSpec((_BM, D), lambda m: (m, 0))
    oshape = jax.ShapeDtypeStruct((N_NODES, D), jnp.float32)
    return pl.pallas_call(
        _proj_body,
        grid=(_GRID,),
        in_specs=[xspec, wspec, bspec] * 3,
        out_specs=[ospec] * 3,
        out_shape=[oshape] * 3,
    )(xc, Wc, bc.reshape(1, D), xd, Wd, bd.reshape(1, D),
      xs, Ws, bs.reshape(1, D))


def _conv_call(relu, agg_a, cnt_a, xdst_a, wl_a, bl_a, wr_a,
               agg_b, cnt_b, xdst_b, wl_b, bl_b, wr_b):
    aspec = pl.BlockSpec((_BM, D), lambda m: (m, 0))
    cspec = pl.BlockSpec((_BM, CW), lambda m: (m, 0))
    wspec = pl.BlockSpec((D, D), lambda m: (0, 0))
    bspec = pl.BlockSpec((1, D), lambda m: (0, 0))
    oshape = jax.ShapeDtypeStruct((N_NODES, D), jnp.float32)
    return pl.pallas_call(
        functools.partial(_conv_body, relu),
        grid=(_GRID,),
        in_specs=[aspec, cspec, aspec, wspec, bspec, wspec] * 2,
        out_specs=[aspec] * 2,
        out_shape=[oshape] * 2,
    )(agg_a, cnt_a, xdst_a, wl_a, bl_a.reshape(1, D), wr_a,
      agg_b, cnt_b, xdst_b, wl_b, bl_b.reshape(1, D), wr_b)


def kernel(x_chemical, x_disease, x_side_effect, edge_index_treats,
           edge_index_rev_treats,
           Wp_c, bp_c, Wp_d, bp_d, Wp_s, bp_s,
           Wl1_td, bl1_td, Wr1_td, Wl1_dc, bl1_dc, Wr1_dc,
           Wl2_td, bl2_td, Wr2_td, Wl2_dc, bl2_dc, Wr2_dc):
    src_td = edge_index_treats[0]
    dst_td = edge_index_treats[1]
    src_dc = edge_index_rev_treats[0]
    dst_dc = edge_index_rev_treats[1]
    zeros64 = jnp.zeros((N_NODES, D), jnp.float32)
    zeros_cw = jnp.zeros((N_NODES, CW), jnp.float32)
    ones = jnp.ones((CHUNK, CW), jnp.float32)

    xc, xd, s1 = _proj_call(x_chemical, Wp_c, bp_c, x_disease, Wp_d, bp_d,
                            x_side_effect, Wp_s, bp_s)

    agg_td, agg_dc, cnt_td, cnt_dc = _make_sc_agg(True)(
        src_td, dst_td, xc, src_dc, dst_dc, xd, zeros64, zeros_cw, ones)

    d1, c1 = _conv_call(True,
                        agg_td, cnt_td, xd, Wl1_td, bl1_td, Wr1_td,
                        agg_dc, cnt_dc, xc, Wl1_dc, bl1_dc, Wr1_dc)

    agg2_td, agg2_dc = _make_sc_agg(False)(
        src_td, dst_td, c1, src_dc, dst_dc, d1, zeros64, zeros_cw, ones)

    d2, c2 = _conv_call(False,
                        agg2_td, cnt_td, d1, Wl2_td, bl2_td, Wr2_td,
                        agg2_dc, cnt_dc, c1, Wl2_dc, bl2_dc, Wr2_dc)

    return c2, d2, s1


# speedup vs baseline: 25.2452x; 1.0100x over previous
import functools

import jax
import jax.numpy as jnp
from jax import lax
from jax.experimental import pallas as pl
from jax.experimental.pallas import tpu as pltpu
from jax.experimental.pallas import tpu_sc as plsc

N_NODES = 10000
IN_DIM = 128
D = 64
CW = 8
E = 640000

NC = 2
NS = 16
EPW = E // NS
CHUNK = 128
_G = 8
_NGROUP = 39
TAIL = EPW - _NGROUP * _G * CHUNK

_ZROWS = [640] * 15 + [400]
_ZOFF = [640 * i for i in range(16)]

_SC_PARAMS = pltpu.CompilerParams(use_tc_tiling_on_sc=False)
_MESH = dict(core_axis_name="c", subcore_axis_name="s",
             num_cores=NC, num_subcores=NS)


def _sc_agg_body(with_counts, src_a, dst_a, tab_a, src_b, dst_b, tab_b,
                 zeros64, zeros_cw, ones_hbm, *refs):
    if with_counts:
        agg_a_out, agg_b_out, cnt_a_out, cnt_b_out = refs[:4]
        refs = refs[4:]
    else:
        agg_a_out, agg_b_out = refs[:2]
        cnt_a_out = cnt_b_out = None
        refs = refs[2:]
    isx = refs[0:_G]
    idx = refs[_G:2 * _G]
    rws = refs[2 * _G:3 * _G]
    isx_t, idx_t, rws_t = refs[3 * _G:3 * _G + 3]
    refs = refs[3 * _G + 3:]
    if with_counts:
        ones_v, acc, cacc, sem_i, sem_g, sem_s = refs
    else:
        acc, sem_i, sem_g, sem_s = refs
        cacc = None

    c = lax.axis_index("c")
    s = lax.axis_index("s")
    base = s * EPW

    def run_direction(src_hbm, dst_hbm, tab_hbm, agg_out, cnt_out):
        for t in range(NS):
            @pl.when(s == t)
            def _():
                sl = pl.ds(_ZOFF[t], _ZROWS[t])
                pltpu.sync_copy(zeros64.at[sl], acc.at[sl])
                if with_counts:
                    pltpu.sync_copy(zeros_cw.at[sl], cacc.at[sl])
        if with_counts:
            pltpu.sync_copy(ones_hbm, ones_v)
        plsc.subcore_barrier()

        def drain_scatters(ks):
            for k in ks:
                pltpu.make_async_copy(rws[k], acc.at[idx[k]], sem_s).wait()
                if with_counts:
                    pltpu.make_async_copy(ones_v, cacc.at[idx[k]],
                                          sem_s).wait()

        def run_set(g, ks):
            @pl.when(g > 0)
            def _():
                drain_scatters(ks)
            dsi = []
            for k in ks:
                sl = pl.ds(base + (g * _G + k) * CHUNK, CHUNK)
                dsi.append(pltpu.async_copy(src_hbm.at[sl], isx[k], sem_i))
                dsi.append(pltpu.async_copy(dst_hbm.at[sl], idx[k], sem_i))
            dsg = []
            for j, k in enumerate(ks):
                dsi[2 * j].wait()
                dsi[2 * j + 1].wait()
                dsg.append(pltpu.async_copy(tab_hbm.at[isx[k]], rws[k],
                                            sem_g))
            for j, k in enumerate(ks):
                dsg[j].wait()
                pltpu.async_copy(rws[k], acc.at[idx[k]], sem_s, add=True)
                if with_counts:
                    pltpu.async_copy(ones_v, cacc.at[idx[k]], sem_s,
                                     add=True)

        half = _G // 2
        set0 = list(range(half))
        set1 = list(range(half, _G))

        def group(g, _):
            run_set(g, set0)
            run_set(g, set1)
            return 0

        lax.fori_loop(0, _NGROUP, group, 0)
        drain_scatters(set0)
        drain_scatters(set1)

        sl = pl.ds(base + _NGROUP * _G * CHUNK, TAIL)
        pltpu.sync_copy(src_hbm.at[sl], isx_t)
        pltpu.sync_copy(dst_hbm.at[sl], idx_t)
        pltpu.async_copy(tab_hbm.at[isx_t], rws_t, sem_g).wait()
        pltpu.async_copy(rws_t, acc.at[idx_t], sem_s, add=True)
        if with_counts:
            pltpu.async_copy(ones_v.at[pl.ds(0, TAIL)], cacc.at[idx_t],
                             sem_s, add=True)
            pltpu.make_async_copy(ones_v.at[pl.ds(0, TAIL)],
                                  cacc.at[idx_t], sem_s).wait()
        pltpu.make_async_copy(rws_t, acc.at[idx_t], sem_s).wait()
        plsc.subcore_barrier()

        for t in range(NS):
            @pl.when(s == t)
            def _():
                sl = pl.ds(_ZOFF[t], _ZROWS[t])
                pltpu.sync_copy(acc.at[sl], agg_out.at[sl])
                if with_counts:
                    pltpu.sync_copy(cacc.at[sl], cnt_out.at[sl])

    @pl.when(c == 0)
    def _():
        run_direction(src_a, dst_a, tab_a, agg_a_out, cnt_a_out)

    @pl.when(c == 1)
    def _():
        run_direction(src_b, dst_b, tab_b, agg_b_out, cnt_b_out)


def _make_sc_agg(with_counts):
    mesh = plsc.VectorSubcoreMesh(**_MESH)
    out_type = [
        jax.ShapeDtypeStruct((N_NODES, D), jnp.float32),
        jax.ShapeDtypeStruct((N_NODES, D), jnp.float32),
    ]
    if with_counts:
        out_type += [
            jax.ShapeDtypeStruct((N_NODES, CW), jnp.float32),
            jax.ShapeDtypeStruct((N_NODES, CW), jnp.float32),
        ]
    scratch = ([pltpu.VMEM((CHUNK,), jnp.int32)] * (2 * _G) +
               [pltpu.VMEM((CHUNK, D), jnp.float32)] * _G +
               [pltpu.VMEM((TAIL,), jnp.int32)] * 2 +
               [pltpu.VMEM((TAIL, D), jnp.float32)])
    if with_counts:
        scratch += [pltpu.VMEM((CHUNK, CW), jnp.float32)]
    scratch += [pltpu.VMEM_SHARED((N_NODES, D), jnp.float32)]
    if with_counts:
        scratch += [pltpu.VMEM_SHARED((N_NODES, CW), jnp.float32)]
    scratch += [pltpu.SemaphoreType.DMA] * 3
    return pl.kernel(
        functools.partial(_sc_agg_body, with_counts),
        out_type=tuple(out_type),
        mesh=mesh,
        scratch_types=tuple(scratch),
        compiler_params=_SC_PARAMS,
    )


def _proj_body(xc, wc, bc, xd, wd, bd, xs, ws, bs, oc, od, os_):
    dn = (((1,), (1,)), ((), ()))
    oc[...] = lax.dot_general(xc[...], wc[...], dn,
                              preferred_element_type=jnp.float32) + bc[...]
    od[...] = lax.dot_general(xd[...], wd[...], dn,
                              preferred_element_type=jnp.float32) + bd[...]
    os_[...] = jnp.maximum(
        lax.dot_general(xs[...], ws[...], dn,
                        preferred_element_type=jnp.float32) + bs[...], 0.0)


def _conv_body(relu, agg_a, cnt_a, xdst_a, wl_a, bl_a, wr_a,
               agg_b, cnt_b, xdst_b, wl_b, bl_b, wr_b, oa, ob):
    dn = (((1,), (1,)), ((), ()))

    def one(agg, cnt, xdst, wl, bl, wr, out):
        inv = 1.0 / jnp.maximum(cnt[:, 0], 1.0)
        mean = agg[...] * inv[:, None]
        r = (lax.dot_general(mean, wl[...], dn,
                             preferred_element_type=jnp.float32) + bl[...] +
             lax.dot_general(xdst[...], wr[...], dn,
                             preferred_element_type=jnp.float32))
        out[...] = jnp.maximum(r, 0.0) if relu else r

    one(agg_a, cnt_a, xdst_a, wl_a, bl_a, wr_a, oa)
    one(agg_b, cnt_b, xdst_b, wl_b, bl_b, wr_b, ob)


_BM = 1000
_GRID = N_NODES // _BM


def _proj_call(xc, Wc, bc, xd, Wd, bd, xs, Ws, bs):
    xspec = pl.BlockSpec((_BM, IN_DIM), lambda m: (m, 0))
    wspec = pl.BlockSpec((D, IN_DIM), lambda m: (0, 0))
    bspec = pl.BlockSpec((1, D), lambda m: (0, 0))
    ospec = pl.BlockSpec((_BM, D), lambda m: (m, 0))
    oshape = jax.ShapeDtypeStruct((N_NODES, D), jnp.float32)
    return pl.pallas_call(
        _proj_body,
        grid=(_GRID,),
        in_specs=[xspec, wspec, bspec] * 3,
        out_specs=[ospec] * 3,
        out_shape=[oshape] * 3,
    )(xc, Wc, bc.reshape(1, D), xd, Wd, bd.reshape(1, D),
      xs, Ws, bs.reshape(1, D))


def _conv_call(relu, agg_a, cnt_a, xdst_a, wl_a, bl_a, wr_a,
               agg_b, cnt_b, xdst_b, wl_b, bl_b, wr_b):
    aspec = pl.BlockSpec((_BM, D), lambda m: (m, 0))
    cspec = pl.BlockSpec((_BM, CW), lambda m: (m, 0))
    wspec = pl.BlockSpec((D, D), lambda m: (0, 0))
    bspec = pl.BlockSpec((1, D), lambda m: (0, 0))
    oshape = jax.ShapeDtypeStruct((N_NODES, D), jnp.float32)
    return pl.pallas_call(
        functools.partial(_conv_body, relu),
        grid=(_GRID,),
        in_specs=[aspec, cspec, aspec, wspec, bspec, wspec] * 2,
        out_specs=[aspec] * 2,
        out_shape=[oshape] * 2,
    )(agg_a, cnt_a, xdst_a, wl_a, bl_a.reshape(1, D), wr_a,
      agg_b, cnt_b, xdst_b, wl_b, bl_b.reshape(1, D), wr_b)


def kernel(x_chemical, x_disease, x_side_effect, edge_index_treats,
           edge_index_rev_treats,
           Wp_c, bp_c, Wp_d, bp_d, Wp_s, bp_s,
           Wl1_td, bl1_td, Wr1_td, Wl1_dc, bl1_dc, Wr1_dc,
           Wl2_td, bl2_td, Wr2_td, Wl2_dc, bl2_dc, Wr2_dc):
    src_td = edge_index_treats[0]
    dst_td = edge_index_treats[1]
    src_dc = edge_index_rev_treats[0]
    dst_dc = edge_index_rev_treats[1]
    zeros64 = jnp.zeros((N_NODES, D), jnp.float32)
    zeros_cw = jnp.zeros((N_NODES, CW), jnp.float32)
    ones = jnp.ones((CHUNK, CW), jnp.float32)

    xc, xd, s1 = _proj_call(x_chemical, Wp_c, bp_c, x_disease, Wp_d, bp_d,
                            x_side_effect, Wp_s, bp_s)

    agg_td, agg_dc, cnt_td, cnt_dc = _make_sc_agg(True)(
        src_td, dst_td, xc, src_dc, dst_dc, xd, zeros64, zeros_cw, ones)

    d1, c1 = _conv_call(True,
                        agg_td, cnt_td, xd, Wl1_td, bl1_td, Wr1_td,
                        agg_dc, cnt_dc, xc, Wl1_dc, bl1_dc, Wr1_dc)

    agg2_td, agg2_dc = _make_sc_agg(False)(
        src_td, dst_td, c1, src_dc, dst_dc, d1, zeros64, zeros_cw, ones)

    d2, c2 = _conv_call(False,
                        agg2_td, cnt_td, d1, Wl2_td, bl2_td, Wr2_td,
                        agg2_dc, cnt_dc, c1, Wl2_dc, bl2_dc, Wr2_dc)

    return c2, d2, s1


# speedup vs baseline: 26.0225x; 1.0308x over previous
import functools

import jax
import jax.numpy as jnp
from jax import lax
from jax.experimental import pallas as pl
from jax.experimental.pallas import tpu as pltpu
from jax.experimental.pallas import tpu_sc as plsc

N_NODES = 10000
IN_DIM = 128
D = 64
CW = 8
E = 640000

NC = 2
NS = 16
EPW = E // NS
CHUNK = 128
_G = 8
_NGROUP = 39
TAIL = EPW - _NGROUP * _G * CHUNK

_ZROWS = [640] * 15 + [400]
_ZOFF = [640 * i for i in range(16)]

_SC_PARAMS = pltpu.CompilerParams(use_tc_tiling_on_sc=False)
_MESH = dict(core_axis_name="c", subcore_axis_name="s",
             num_cores=NC, num_subcores=NS)


def _sc_agg_body(with_counts, src_a, dst_a, tab_a, src_b, dst_b, tab_b,
                 zeros64, zeros_cw, ones_hbm, *refs):
    if with_counts:
        agg_a_out, agg_b_out, cnt_a_out, cnt_b_out = refs[:4]
        refs = refs[4:]
    else:
        agg_a_out, agg_b_out = refs[:2]
        cnt_a_out = cnt_b_out = None
        refs = refs[2:]
    isx = refs[0:_G]
    idx = refs[_G:2 * _G]
    rws = refs[2 * _G:3 * _G]
    isx_t, idx_t, rws_t = refs[3 * _G:3 * _G + 3]
    refs = refs[3 * _G + 3:]
    if with_counts:
        ones_v, acc, cacc, sem_i, sem_g, sem_s = refs
    else:
        acc, sem_i, sem_g, sem_s = refs
        cacc = None

    c = lax.axis_index("c")
    s = lax.axis_index("s")
    base = s * EPW

    def run_direction(src_hbm, dst_hbm, tab_hbm, agg_out, cnt_out):
        for t in range(NS):
            @pl.when(s == t)
            def _():
                sl = pl.ds(_ZOFF[t], _ZROWS[t])
                pltpu.sync_copy(zeros64.at[sl], acc.at[sl])
                if with_counts:
                    pltpu.sync_copy(zeros_cw.at[sl], cacc.at[sl])
        if with_counts:
            pltpu.sync_copy(ones_hbm, ones_v)
        plsc.subcore_barrier()

        def drain_scatters(ks):
            for k in ks:
                pltpu.make_async_copy(rws[k], acc.at[idx[k]], sem_s).wait()
                if with_counts:
                    pltpu.make_async_copy(ones_v, cacc.at[idx[k]],
                                          sem_s).wait()

        def run_set(g, ks):
            @pl.when(g > 0)
            def _():
                drain_scatters(ks)
            dsi = []
            for k in ks:
                sl = pl.ds(base + (g * _G + k) * CHUNK, CHUNK)
                dsi.append(pltpu.async_copy(src_hbm.at[sl], isx[k], sem_i))
                dsi.append(pltpu.async_copy(dst_hbm.at[sl], idx[k], sem_i))
            dsg = []
            for j, k in enumerate(ks):
                dsi[2 * j].wait()
                dsi[2 * j + 1].wait()
                dsg.append(pltpu.async_copy(tab_hbm.at[isx[k]], rws[k],
                                            sem_g))
            for j, k in enumerate(ks):
                dsg[j].wait()
                pltpu.async_copy(rws[k], acc.at[idx[k]], sem_s, add=True)
                if with_counts:
                    pltpu.async_copy(ones_v, cacc.at[idx[k]], sem_s,
                                     add=True)

        half = _G // 2
        set0 = list(range(half))
        set1 = list(range(half, _G))

        def group(g, _):
            run_set(g, set0)
            run_set(g, set1)
            return 0

        lax.fori_loop(0, _NGROUP, group, 0)
        drain_scatters(set0)
        drain_scatters(set1)

        sl = pl.ds(base + _NGROUP * _G * CHUNK, TAIL)
        pltpu.sync_copy(src_hbm.at[sl], isx_t)
        pltpu.sync_copy(dst_hbm.at[sl], idx_t)
        pltpu.async_copy(tab_hbm.at[isx_t], rws_t, sem_g).wait()
        pltpu.async_copy(rws_t, acc.at[idx_t], sem_s, add=True)
        if with_counts:
            pltpu.async_copy(ones_v.at[pl.ds(0, TAIL)], cacc.at[idx_t],
                             sem_s, add=True)
            pltpu.make_async_copy(ones_v.at[pl.ds(0, TAIL)],
                                  cacc.at[idx_t], sem_s).wait()
        pltpu.make_async_copy(rws_t, acc.at[idx_t], sem_s).wait()
        plsc.subcore_barrier()

        for t in range(NS):
            @pl.when(s == t)
            def _():
                sl = pl.ds(_ZOFF[t], _ZROWS[t])
                pltpu.sync_copy(acc.at[sl], agg_out.at[sl])
                if with_counts:
                    pltpu.sync_copy(cacc.at[sl], cnt_out.at[sl])

    @pl.when(c == 0)
    def _():
        run_direction(src_a, dst_a, tab_a, agg_a_out, cnt_a_out)

    @pl.when(c == 1)
    def _():
        run_direction(src_b, dst_b, tab_b, agg_b_out, cnt_b_out)


def _make_sc_agg(with_counts):
    mesh = plsc.VectorSubcoreMesh(**_MESH)
    out_type = [
        jax.ShapeDtypeStruct((N_NODES, D), jnp.float32),
        jax.ShapeDtypeStruct((N_NODES, D), jnp.float32),
    ]
    if with_counts:
        out_type += [
            jax.ShapeDtypeStruct((N_NODES, CW), jnp.float32),
            jax.ShapeDtypeStruct((N_NODES, CW), jnp.float32),
        ]
    scratch = ([pltpu.VMEM((CHUNK,), jnp.int32)] * (2 * _G) +
               [pltpu.VMEM((CHUNK, D), jnp.float32)] * _G +
               [pltpu.VMEM((TAIL,), jnp.int32)] * 2 +
               [pltpu.VMEM((TAIL, D), jnp.float32)])
    if with_counts:
        scratch += [pltpu.VMEM((CHUNK, CW), jnp.float32)]
    scratch += [pltpu.VMEM_SHARED((N_NODES, D), jnp.float32)]
    if with_counts:
        scratch += [pltpu.VMEM_SHARED((N_NODES, CW), jnp.float32)]
    scratch += [pltpu.SemaphoreType.DMA] * 3
    return pl.kernel(
        functools.partial(_sc_agg_body, with_counts),
        out_type=tuple(out_type),
        mesh=mesh,
        scratch_types=tuple(scratch),
        compiler_params=_SC_PARAMS,
    )


_N2 = N_NODES // 2
_D2 = 2 * D
_BM = 1000
_GRID = _N2 // _BM


def _proj_body(xc, wc, bc, xd, wd, bd, xs, ws, bs, oc, od, os_):
    dn2 = (((1,), (0,)), ((), ()))
    dnt = (((1,), (1,)), ((), ()))
    oc[...] = lax.dot_general(xc[...], wc[...], dn2,
                              preferred_element_type=jnp.float32) + bc[...]
    od[...] = lax.dot_general(xd[...], wd[...], dn2,
                              preferred_element_type=jnp.float32) + bd[...]
    os_[...] = jnp.maximum(
        lax.dot_general(xs[...], ws[...], dnt,
                        preferred_element_type=jnp.float32) + bs[...], 0.0)


def _conv_body(relu, agg_a, inv_a, xdst_a, wl_a, bl_a, wr_a,
               agg_b, inv_b, xdst_b, wl_b, bl_b, wr_b, oa, ob):
    dn2 = (((1,), (0,)), ((), ()))

    def one(agg, inv, xdst, wl, bl, wr, out):
        mean = agg[...] * inv[...]
        r = (lax.dot_general(mean, wl[...], dn2,
                             preferred_element_type=jnp.float32) + bl[...] +
             lax.dot_general(xdst[...], wr[...], dn2,
                             preferred_element_type=jnp.float32))
        out[...] = jnp.maximum(r, 0.0) if relu else r

    one(agg_a, inv_a, xdst_a, wl_a, bl_a, wr_a, oa)
    one(agg_b, inv_b, xdst_b, wl_b, bl_b, wr_b, ob)


def _proj_call(xc2, Pc, bc, xd2, Pd, bd, xs, Ws, bs):
    xspec = pl.BlockSpec((_BM, 2 * IN_DIM), lambda m: (m, 0))
    pspec = pl.BlockSpec((2 * IN_DIM, _D2), lambda m: (0, 0))
    b2spec = pl.BlockSpec((1, _D2), lambda m: (0, 0))
    o2spec = pl.BlockSpec((_BM, _D2), lambda m: (m, 0))
    sspec = pl.BlockSpec((2 * _BM, IN_DIM), lambda m: (m, 0))
    wsspec = pl.BlockSpec((D, IN_DIM), lambda m: (0, 0))
    bsspec = pl.BlockSpec((1, D), lambda m: (0, 0))
    osspec = pl.BlockSpec((2 * _BM, D), lambda m: (m, 0))
    return pl.pallas_call(
        _proj_body,
        grid=(_GRID,),
        in_specs=[xspec, pspec, b2spec, xspec, pspec, b2spec,
                  sspec, wsspec, bsspec],
        out_specs=[o2spec, o2spec, osspec],
        out_shape=[jax.ShapeDtypeStruct((_N2, _D2), jnp.float32)] * 2 +
                  [jax.ShapeDtypeStruct((N_NODES, D), jnp.float32)],
    )(xc2, Pc, bc, xd2, Pd, bd, xs, Ws, bs.reshape(1, D))


def _conv_call(relu, agg_a, inv_a, xdst_a, wl_a, bl_a, wr_a,
               agg_b, inv_b, xdst_b, wl_b, bl_b, wr_b):
    aspec = pl.BlockSpec((_BM, _D2), lambda m: (m, 0))
    wspec = pl.BlockSpec((_D2, _D2), lambda m: (0, 0))
    bspec = pl.BlockSpec((1, _D2), lambda m: (0, 0))
    oshape = jax.ShapeDtypeStruct((_N2, _D2), jnp.float32)
    return pl.pallas_call(
        functools.partial(_conv_body, relu),
        grid=(_GRID,),
        in_specs=[aspec, aspec, aspec, wspec, bspec, wspec] * 2,
        out_specs=[aspec] * 2,
        out_shape=[oshape] * 2,
    )(agg_a, inv_a, xdst_a, wl_a, bl_a, wr_a,
      agg_b, inv_b, xdst_b, wl_b, bl_b, wr_b)


def _blockdiag2(Wt):
    k, n = Wt.shape
    z = jnp.zeros((k, n), jnp.float32)
    return jnp.concatenate([
        jnp.concatenate([Wt, z], axis=1),
        jnp.concatenate([z, Wt], axis=1),
    ], axis=0)


def _inv128(cnt):
    inv = 1.0 / jnp.maximum(cnt[:, 0], 1.0)
    return jnp.repeat(inv.reshape(_N2, 2), D, axis=1)


def kernel(x_chemical, x_disease, x_side_effect, edge_index_treats,
           edge_index_rev_treats,
           Wp_c, bp_c, Wp_d, bp_d, Wp_s, bp_s,
           Wl1_td, bl1_td, Wr1_td, Wl1_dc, bl1_dc, Wr1_dc,
           Wl2_td, bl2_td, Wr2_td, Wl2_dc, bl2_dc, Wr2_dc):
    src_td = edge_index_treats[0]
    dst_td = edge_index_treats[1]
    src_dc = edge_index_rev_treats[0]
    dst_dc = edge_index_rev_treats[1]
    zeros64 = jnp.zeros((N_NODES, D), jnp.float32)
    zeros_cw = jnp.zeros((N_NODES, CW), jnp.float32)
    ones = jnp.ones((CHUNK, CW), jnp.float32)

    def bd2(b):
        return jnp.concatenate([b, b]).reshape(1, _D2)

    xc2, xd2, s1 = _proj_call(
        x_chemical.reshape(_N2, 2 * IN_DIM), _blockdiag2(Wp_c.T), bd2(bp_c),
        x_disease.reshape(_N2, 2 * IN_DIM), _blockdiag2(Wp_d.T), bd2(bp_d),
        x_side_effect, Wp_s, bp_s)

    sc1 = _make_sc_agg(True)
    agg_td, agg_dc, cnt_td, cnt_dc = sc1(
        src_td, dst_td, xc2.reshape(N_NODES, D),
        src_dc, dst_dc, xd2.reshape(N_NODES, D),
        zeros64, zeros_cw, ones)
    inv_td = _inv128(cnt_td)
    inv_dc = _inv128(cnt_dc)

    d1, c1 = _conv_call(
        True,
        agg_td.reshape(_N2, _D2), inv_td, xd2,
        _blockdiag2(Wl1_td.T), bd2(bl1_td), _blockdiag2(Wr1_td.T),
        agg_dc.reshape(_N2, _D2), inv_dc, xc2,
        _blockdiag2(Wl1_dc.T), bd2(bl1_dc), _blockdiag2(Wr1_dc.T))

    agg2_td, agg2_dc = _make_sc_agg(False)(
        src_td, dst_td, c1.reshape(N_NODES, D),
        src_dc, dst_dc, d1.reshape(N_NODES, D),
        zeros64, zeros_cw, ones)

    d2, c2 = _conv_call(
        False,
        agg2_td.reshape(_N2, _D2), inv_td, d1,
        _blockdiag2(Wl2_td.T), bd2(bl2_td), _blockdiag2(Wr2_td.T),
        agg2_dc.reshape(_N2, _D2), inv_dc, c1,
        _blockdiag2(Wl2_dc.T), bd2(bl2_dc), _blockdiag2(Wr2_dc.T))

    return c2.reshape(N_NODES, D), d2.reshape(N_NODES, D), s1


# speedup vs baseline: 26.0373x; 1.0006x over previous
import functools

import jax
import jax.numpy as jnp
from jax import lax
from jax.experimental import pallas as pl
from jax.experimental.pallas import tpu as pltpu
from jax.experimental.pallas import tpu_sc as plsc

N_NODES = 10000
IN_DIM = 128
D = 64
CW = 8
E = 640000

NC = 2
NS = 16
EPW = E // NS
CHUNK = 128
_G = 8
_NGROUP = 39
TAIL = EPW - _NGROUP * _G * CHUNK

_ZROWS = [640] * 15 + [400]
_ZOFF = [640 * i for i in range(16)]

_SC_PARAMS = pltpu.CompilerParams(use_tc_tiling_on_sc=False)
_MESH = dict(core_axis_name="c", subcore_axis_name="s",
             num_cores=NC, num_subcores=NS)


def _sc_agg_body(with_counts, src_a, dst_a, tab_a, src_b, dst_b, tab_b,
                 zeros64, zeros_cw, ones_hbm, *refs):
    if with_counts:
        agg_a_out, agg_b_out, cnt_a_out, cnt_b_out = refs[:4]
        refs = refs[4:]
    else:
        agg_a_out, agg_b_out = refs[:2]
        cnt_a_out = cnt_b_out = None
        refs = refs[2:]
    isx = refs[0:_G]
    idx = refs[_G:2 * _G]
    rws = refs[2 * _G:3 * _G]
    isx_t, idx_t, rws_t = refs[3 * _G:3 * _G + 3]
    refs = refs[3 * _G + 3:]
    if with_counts:
        ones_v, acc, cacc, sem_i, sem_g, sem_s = refs
    else:
        acc, sem_i, sem_g, sem_s = refs
        cacc = None

    c = lax.axis_index("c")
    s = lax.axis_index("s")
    base = s * EPW

    def run_direction(src_hbm, dst_hbm, tab_hbm, agg_out, cnt_out):
        for t in range(NS):
            @pl.when(s == t)
            def _():
                sl = pl.ds(_ZOFF[t], _ZROWS[t])
                pltpu.sync_copy(zeros64.at[sl], acc.at[sl])
                if with_counts:
                    pltpu.sync_copy(zeros_cw.at[sl], cacc.at[sl])
        if with_counts:
            pltpu.sync_copy(ones_hbm, ones_v)
        plsc.subcore_barrier()

        def drain_scatters(ks):
            for k in ks:
                pltpu.make_async_copy(rws[k], acc.at[idx[k]], sem_s).wait()
                if with_counts:
                    pltpu.make_async_copy(ones_v, cacc.at[idx[k]],
                                          sem_s).wait()

        def run_set(g, ks):
            @pl.when(g > 0)
            def _():
                drain_scatters(ks)
            dsi = []
            for k in ks:
                sl = pl.ds(base + (g * _G + k) * CHUNK, CHUNK)
                dsi.append(pltpu.async_copy(src_hbm.at[sl], isx[k], sem_i))
                dsi.append(pltpu.async_copy(dst_hbm.at[sl], idx[k], sem_i))
            dsg = []
            for j, k in enumerate(ks):
                dsi[2 * j].wait()
                dsi[2 * j + 1].wait()
                dsg.append(pltpu.async_copy(tab_hbm.at[isx[k]], rws[k],
                                            sem_g))
            for j, k in enumerate(ks):
                dsg[j].wait()
                pltpu.async_copy(rws[k], acc.at[idx[k]], sem_s, add=True)
                if with_counts:
                    pltpu.async_copy(ones_v, cacc.at[idx[k]], sem_s,
                                     add=True)

        half = _G // 2
        set0 = list(range(half))
        set1 = list(range(half, _G))

        def group(g, _):
            run_set(g, set0)
            run_set(g, set1)
            return 0

        lax.fori_loop(0, _NGROUP, group, 0)
        drain_scatters(set0)
        drain_scatters(set1)

        sl = pl.ds(base + _NGROUP * _G * CHUNK, TAIL)
        pltpu.sync_copy(src_hbm.at[sl], isx_t)
        pltpu.sync_copy(dst_hbm.at[sl], idx_t)
        pltpu.async_copy(tab_hbm.at[isx_t], rws_t, sem_g).wait()
        pltpu.async_copy(rws_t, acc.at[idx_t], sem_s, add=True)
        if with_counts:
            pltpu.async_copy(ones_v.at[pl.ds(0, TAIL)], cacc.at[idx_t],
                             sem_s, add=True)
            pltpu.make_async_copy(ones_v.at[pl.ds(0, TAIL)],
                                  cacc.at[idx_t], sem_s).wait()
        pltpu.make_async_copy(rws_t, acc.at[idx_t], sem_s).wait()
        plsc.subcore_barrier()

        for t in range(NS):
            @pl.when(s == t)
            def _():
                sl = pl.ds(_ZOFF[t], _ZROWS[t])
                pltpu.sync_copy(acc.at[sl], agg_out.at[sl])
                if with_counts:
                    pltpu.sync_copy(cacc.at[sl], cnt_out.at[sl])

    @pl.when(c == 0)
    def _():
        run_direction(src_a, dst_a, tab_a, agg_a_out, cnt_a_out)

    @pl.when(c == 1)
    def _():
        run_direction(src_b, dst_b, tab_b, agg_b_out, cnt_b_out)


def _make_sc_agg(with_counts):
    mesh = plsc.VectorSubcoreMesh(**_MESH)
    out_type = [
        jax.ShapeDtypeStruct((N_NODES, D), jnp.float32),
        jax.ShapeDtypeStruct((N_NODES, D), jnp.float32),
    ]
    if with_counts:
        out_type += [
            jax.ShapeDtypeStruct((N_NODES, CW), jnp.float32),
            jax.ShapeDtypeStruct((N_NODES, CW), jnp.float32),
        ]
    scratch = ([pltpu.VMEM((CHUNK,), jnp.int32)] * (2 * _G) +
               [pltpu.VMEM((CHUNK, D), jnp.float32)] * _G +
               [pltpu.VMEM((TAIL,), jnp.int32)] * 2 +
               [pltpu.VMEM((TAIL, D), jnp.float32)])
    if with_counts:
        scratch += [pltpu.VMEM((CHUNK, CW), jnp.float32)]
    scratch += [pltpu.VMEM_SHARED((N_NODES, D), jnp.float32)]
    if with_counts:
        scratch += [pltpu.VMEM_SHARED((N_NODES, CW), jnp.float32)]
    scratch += [pltpu.SemaphoreType.DMA] * 3
    return pl.kernel(
        functools.partial(_sc_agg_body, with_counts),
        out_type=tuple(out_type),
        mesh=mesh,
        scratch_types=tuple(scratch),
        compiler_params=_SC_PARAMS,
    )


_N2 = N_NODES // 2
_D2 = 2 * D
_BM = 1000
_GRID = _N2 // _BM


def _sc_cnt_body(dst_a, dst_b, zeros_cw, ones_hbm, *refs):
    cnt_a_out, cnt_b_out = refs[:2]
    refs = refs[2:]
    idx = refs[0:_G]
    idx_t, ones_v, cacc, sem_i, sem_s = refs[_G:]

    c = lax.axis_index("c")
    s = lax.axis_index("s")
    base = s * EPW

    def run_direction(dst_hbm, cnt_out):
        for t in range(NS):
            @pl.when(s == t)
            def _():
                sl = pl.ds(_ZOFF[t], _ZROWS[t])
                pltpu.sync_copy(zeros_cw.at[sl], cacc.at[sl])
        pltpu.sync_copy(ones_hbm, ones_v)
        plsc.subcore_barrier()

        def drain_scatters(ks):
            for k in ks:
                pltpu.make_async_copy(ones_v, cacc.at[idx[k]], sem_s).wait()

        def run_set(g, ks):
            @pl.when(g > 0)
            def _():
                drain_scatters(ks)
            dsi = []
            for k in ks:
                sl = pl.ds(base + (g * _G + k) * CHUNK, CHUNK)
                dsi.append(pltpu.async_copy(dst_hbm.at[sl], idx[k], sem_i))
            for j, k in enumerate(ks):
                dsi[j].wait()
                pltpu.async_copy(ones_v, cacc.at[idx[k]], sem_s, add=True)

        half = _G // 2
        set0 = list(range(half))
        set1 = list(range(half, _G))

        def group(g, _):
            run_set(g, set0)
            run_set(g, set1)
            return 0

        lax.fori_loop(0, _NGROUP, group, 0)
        drain_scatters(set0)
        drain_scatters(set1)

        sl = pl.ds(base + _NGROUP * _G * CHUNK, TAIL)
        pltpu.sync_copy(dst_hbm.at[sl], idx_t)
        pltpu.async_copy(ones_v.at[pl.ds(0, TAIL)], cacc.at[idx_t],
                         sem_s, add=True)
        pltpu.make_async_copy(ones_v.at[pl.ds(0, TAIL)], cacc.at[idx_t],
                              sem_s).wait()
        plsc.subcore_barrier()

        for t in range(NS):
            @pl.when(s == t)
            def _():
                sl = pl.ds(_ZOFF[t], _ZROWS[t])
                pltpu.sync_copy(cacc.at[sl], cnt_out.at[sl])

    @pl.when(c == 0)
    def _():
        run_direction(dst_a, cnt_a_out)

    @pl.when(c == 1)
    def _():
        run_direction(dst_b, cnt_b_out)


def _make_sc_cnt():
    mesh = plsc.VectorSubcoreMesh(**_MESH)
    out_type = (
        jax.ShapeDtypeStruct((N_NODES, CW), jnp.float32),
        jax.ShapeDtypeStruct((N_NODES, CW), jnp.float32),
    )
    scratch = ([pltpu.VMEM((CHUNK,), jnp.int32)] * _G +
               [pltpu.VMEM((TAIL,), jnp.int32)] +
               [pltpu.VMEM((CHUNK, CW), jnp.float32)] +
               [pltpu.VMEM_SHARED((N_NODES, CW), jnp.float32)] +
               [pltpu.SemaphoreType.DMA] * 2)
    return pl.kernel(
        _sc_cnt_body,
        out_type=out_type,
        mesh=mesh,
        scratch_types=tuple(scratch),
        compiler_params=_SC_PARAMS,
    )


def _proj_body(xc, wc, bc, xd, wd, bd, xs, ws, bs, oc, od, os_):
    dn2 = (((1,), (0,)), ((), ()))
    dnt = (((1,), (1,)), ((), ()))
    oc[...] = lax.dot_general(xc[...], wc[...], dn2,
                              preferred_element_type=jnp.float32) + bc[...]
    od[...] = lax.dot_general(xd[...], wd[...], dn2,
                              preferred_element_type=jnp.float32) + bd[...]
    os_[...] = jnp.maximum(
        lax.dot_general(xs[...], ws[...], dnt,
                        preferred_element_type=jnp.float32) + bs[...], 0.0)


def _conv_body(relu, agg_a, inv_a, xdst_a, wl_a, bl_a, wr_a,
               agg_b, inv_b, xdst_b, wl_b, bl_b, wr_b, oa, ob):
    dn2 = (((1,), (0,)), ((), ()))

    def one(agg, inv, xdst, wl, bl, wr, out):
        mean = agg[...] * inv[...]
        r = (lax.dot_general(mean, wl[...], dn2,
                             preferred_element_type=jnp.float32) + bl[...] +
             lax.dot_general(xdst[...], wr[...], dn2,
                             preferred_element_type=jnp.float32))
        out[...] = jnp.maximum(r, 0.0) if relu else r

    one(agg_a, inv_a, xdst_a, wl_a, bl_a, wr_a, oa)
    one(agg_b, inv_b, xdst_b, wl_b, bl_b, wr_b, ob)


def _proj_call(xc2, Pc, bc, xd2, Pd, bd, xs, Ws, bs):
    xspec = pl.BlockSpec((_BM, 2 * IN_DIM), lambda m: (m, 0))
    pspec = pl.BlockSpec((2 * IN_DIM, _D2), lambda m: (0, 0))
    b2spec = pl.BlockSpec((1, _D2), lambda m: (0, 0))
    o2spec = pl.BlockSpec((_BM, _D2), lambda m: (m, 0))
    sspec = pl.BlockSpec((2 * _BM, IN_DIM), lambda m: (m, 0))
    wsspec = pl.BlockSpec((D, IN_DIM), lambda m: (0, 0))
    bsspec = pl.BlockSpec((1, D), lambda m: (0, 0))
    osspec = pl.BlockSpec((2 * _BM, D), lambda m: (m, 0))
    return pl.pallas_call(
        _proj_body,
        grid=(_GRID,),
        in_specs=[xspec, pspec, b2spec, xspec, pspec, b2spec,
                  sspec, wsspec, bsspec],
        out_specs=[o2spec, o2spec, osspec],
        out_shape=[jax.ShapeDtypeStruct((_N2, _D2), jnp.float32)] * 2 +
                  [jax.ShapeDtypeStruct((N_NODES, D), jnp.float32)],
    )(xc2, Pc, bc, xd2, Pd, bd, xs, Ws, bs.reshape(1, D))


def _conv_call(relu, agg_a, inv_a, xdst_a, wl_a, bl_a, wr_a,
               agg_b, inv_b, xdst_b, wl_b, bl_b, wr_b):
    aspec = pl.BlockSpec((_BM, _D2), lambda m: (m, 0))
    wspec = pl.BlockSpec((_D2, _D2), lambda m: (0, 0))
    bspec = pl.BlockSpec((1, _D2), lambda m: (0, 0))
    oshape = jax.ShapeDtypeStruct((_N2, _D2), jnp.float32)
    return pl.pallas_call(
        functools.partial(_conv_body, relu),
        grid=(_GRID,),
        in_specs=[aspec, aspec, aspec, wspec, bspec, wspec] * 2,
        out_specs=[aspec] * 2,
        out_shape=[oshape] * 2,
    )(agg_a, inv_a, xdst_a, wl_a, bl_a, wr_a,
      agg_b, inv_b, xdst_b, wl_b, bl_b, wr_b)


def _blockdiag2(Wt):
    k, n = Wt.shape
    z = jnp.zeros((k, n), jnp.float32)
    return jnp.concatenate([
        jnp.concatenate([Wt, z], axis=1),
        jnp.concatenate([z, Wt], axis=1),
    ], axis=0)


def _inv128(cnt):
    inv = 1.0 / jnp.maximum(cnt[:, 0], 1.0)
    return jnp.repeat(inv.reshape(_N2, 2), D, axis=1)


def kernel(x_chemical, x_disease, x_side_effect, edge_index_treats,
           edge_index_rev_treats,
           Wp_c, bp_c, Wp_d, bp_d, Wp_s, bp_s,
           Wl1_td, bl1_td, Wr1_td, Wl1_dc, bl1_dc, Wr1_dc,
           Wl2_td, bl2_td, Wr2_td, Wl2_dc, bl2_dc, Wr2_dc):
    src_td = edge_index_treats[0]
    dst_td = edge_index_treats[1]
    src_dc = edge_index_rev_treats[0]
    dst_dc = edge_index_rev_treats[1]
    zeros64 = jnp.zeros((N_NODES, D), jnp.float32)
    zeros_cw = jnp.zeros((N_NODES, CW), jnp.float32)
    ones = jnp.ones((CHUNK, CW), jnp.float32)

    def bd2(b):
        return jnp.concatenate([b, b]).reshape(1, _D2)

    cnt_td, cnt_dc = _make_sc_cnt()(dst_td, dst_dc, zeros_cw, ones)
    inv_td = _inv128(cnt_td)
    inv_dc = _inv128(cnt_dc)

    xc2, xd2, s1 = _proj_call(
        x_chemical.reshape(_N2, 2 * IN_DIM), _blockdiag2(Wp_c.T), bd2(bp_c),
        x_disease.reshape(_N2, 2 * IN_DIM), _blockdiag2(Wp_d.T), bd2(bp_d),
        x_side_effect, Wp_s, bp_s)

    sc_agg = _make_sc_agg(False)
    agg_td, agg_dc = sc_agg(
        src_td, dst_td, xc2.reshape(N_NODES, D),
        src_dc, dst_dc, xd2.reshape(N_NODES, D),
        zeros64, zeros_cw, ones)

    d1, c1 = _conv_call(
        True,
        agg_td.reshape(_N2, _D2), inv_td, xd2,
        _blockdiag2(Wl1_td.T), bd2(bl1_td), _blockdiag2(Wr1_td.T),
        agg_dc.reshape(_N2, _D2), inv_dc, xc2,
        _blockdiag2(Wl1_dc.T), bd2(bl1_dc), _blockdiag2(Wr1_dc.T))

    agg2_td, agg2_dc = sc_agg(
        src_td, dst_td, c1.reshape(N_NODES, D),
        src_dc, dst_dc, d1.reshape(N_NODES, D),
        zeros64, zeros_cw, ones)

    d2, c2 = _conv_call(
        False,
        agg2_td.reshape(_N2, _D2), inv_td, d1,
        _blockdiag2(Wl2_td.T), bd2(bl2_td), _blockdiag2(Wr2_td.T),
        agg2_dc.reshape(_N2, _D2), inv_dc, c1,
        _blockdiag2(Wl2_dc.T), bd2(bl2_dc), _blockdiag2(Wr2_dc.T))

    return c2.reshape(N_NODES, D), d2.reshape(N_NODES, D), s1
